# Initial kernel scaffold; baseline (speedup 1.0000x reference)
import jax
import jax.numpy as jnp
from jax.experimental import pallas as pl


def kernel(x, y, z, edge_index, lg_edge_index, params):
    raise NotImplementedError("write your pallas kernel here")


# speedup vs baseline: 1.4911x; 1.4911x over previous
import functools

import jax
import jax.numpy as jnp
from jax import lax
from jax.experimental import pallas as pl
from jax.experimental.pallas import tpu as pltpu
from jax.experimental.pallas import tpu_sc as plsc

_EPS_BN = 1e-5
_EPS_SEG = 1e-6
_C = 128
_NW = 32
_L = 3


@functools.lru_cache(None)
def _stats_call(R, F, BR):
    def body(a_ref, o_ref):
        a = a_ref[...]

        @pl.when(pl.program_id(0) == 0)
        def _():
            o_ref[...] = jnp.zeros_like(o_ref)

        o_ref[0:1, :] += jnp.sum(a, axis=0, keepdims=True)
        o_ref[1:2, :] += jnp.sum(a * a, axis=0, keepdims=True)

    return pl.pallas_call(
        body,
        grid=(R // BR,),
        in_specs=[pl.BlockSpec((BR, F), lambda i: (i, 0))],
        out_specs=pl.BlockSpec((8, F), lambda i: (0, 0)),
        out_shape=jax.ShapeDtypeStruct((8, F), jnp.float32),
    )


def _stats(a):
    R, F = a.shape
    BR = 2000 if R % 2000 == 0 else 1000
    return _stats_call(R, F, BR)(a)


@functools.lru_cache(None)
def _mm_call(R, BR, npieces, Ks, residual):
    nouts = len(Ks)

    def body(*refs):
        it = iter(refs)
        a = [next(it) for _ in range(npieces)]
        st = [next(it) for _ in range(npieces)]
        g = [next(it) for _ in range(npieces)]
        b = [next(it) for _ in range(npieces)]
        W = [[next(it) for _ in range(npieces)] for _ in range(nouts)]
        res = next(it) if residual else None
        outs = [next(it) for _ in range(nouts)]
        acc = [None] * nouts
        inv_r = 1.0 / R
        for j in range(npieces):
            aj = a[j][...]
            mean = st[j][0:1, :] * inv_r
            var = st[j][1:2, :] * inv_r - mean * mean
            xn = (aj - mean) * lax.rsqrt(var + _EPS_BN) * g[j][...] + b[j][...]
            xn = xn * jax.nn.sigmoid(xn)
            for o in range(nouts):
                d = jnp.dot(xn, W[o][j][...], preferred_element_type=jnp.float32)
                acc[o] = d if acc[o] is None else acc[o] + d
        for o in range(nouts):
            val = acc[o]
            if residual and o == 0:
                val = val + res[...]
            outs[o][...] = val

    in_specs = (
        [pl.BlockSpec((BR, 64), lambda i: (i, 0)) for _ in range(npieces)]
        + [pl.BlockSpec((8, 64), lambda i: (0, 0)) for _ in range(npieces)]
        + [pl.BlockSpec((1, 64), lambda i: (0, 0)) for _ in range(2 * npieces)]
        + [pl.BlockSpec((64, K), lambda i: (0, 0))
           for K in Ks for _ in range(npieces)]
        + ([pl.BlockSpec((BR, Ks[0]), lambda i: (i, 0))] if residual else [])
    )
    return pl.pallas_call(
        body,
        grid=(R // BR,),
        in_specs=in_specs,
        out_specs=[pl.BlockSpec((BR, K), lambda i: (i, 0)) for K in Ks],
        out_shape=[jax.ShapeDtypeStruct((R, K), jnp.float32) for K in Ks],
    )


def _mm(pieces, stats, g, b, Ws, residual=None):
    R = pieces[0].shape[0]
    npieces = len(pieces)
    Ks = tuple(int(W.shape[1]) for W in Ws)
    BR = 2000 if R % 2000 == 0 else 1000
    args = list(pieces) + list(stats)
    for j in range(npieces):
        args.append(g[64 * j:64 * (j + 1)].reshape(1, 64))
    for j in range(npieces):
        args.append(b[64 * j:64 * (j + 1)].reshape(1, 64))
    for W in Ws:
        for j in range(npieces):
            args.append(W[64 * j:64 * (j + 1), :])
    if residual is not None:
        args.append(residual)
    out = _mm_call(R, BR, npieces, Ks, residual is not None)(*args)
    return out


@functools.lru_cache(None)
def _combine_call(R, BR, Racc):
    def body(su_ref, acc_ref, o_ref):
        accv = acc_ref[...]
        o_ref[...] = su_ref[...] + accv[:, 0:64] / (accv[:, 64:128] + _EPS_SEG)

    return pl.pallas_call(
        body,
        grid=(R // BR,),
        in_specs=[pl.BlockSpec((BR, 64), lambda i: (i, 0)),
                  pl.BlockSpec((BR, 128), lambda i: (i, 0))],
        out_specs=pl.BlockSpec((BR, 64), lambda i: (i, 0)),
        out_shape=jax.ShapeDtypeStruct((R, 64), jnp.float32),
    )


@functools.lru_cache(None)
def _edge_compute_call(e, write_m):
    nchunk = e // _C
    mesh = plsc.VectorSubcoreMesh(core_axis_name="c", subcore_axis_name="s")
    out_type = [jax.ShapeDtypeStruct((e, 128), jnp.float32)]
    if write_m:
        out_type = [jax.ShapeDtypeStruct((e, 64), jnp.float32)] + out_type
    scratch = [
        pltpu.VMEM((_C,), jnp.int32), pltpu.VMEM((_C,), jnp.int32),
        pltpu.VMEM((_C, 128), jnp.float32), pltpu.VMEM((_C, 128), jnp.float32),
        pltpu.VMEM((_C, 64), jnp.float32), pltpu.VMEM((_C, 64), jnp.float32),
        pltpu.VMEM((_C, 128), jnp.float32),
        pltpu.SemaphoreType.DMA, pltpu.SemaphoreType.DMA,
        pltpu.SemaphoreType.DMA,
    ]

    @functools.partial(pl.kernel, mesh=mesh, out_type=out_type,
                       scratch_types=scratch)
    def k(tbl_src, tbl_dst, ef, src, dst, *rest):
        ri = iter(rest)
        m_out = next(ri) if write_m else None
        c_out = next(ri)
        idxs_v = next(ri)
        idxd_v = next(ri)
        gsrc_v = next(ri)
        gdst_v = next(ri)
        ef_v = next(ri)
        m_v = next(ri)
        co_v = next(ri)
        s1 = next(ri)
        s2 = next(ri)
        s3 = next(ri)
        wid = lax.axis_index("s") * 2 + lax.axis_index("c")
        c0 = wid * nchunk // _NW
        c1 = (wid + 1) * nchunk // _NW

        def chunk(ci, carry):
            base = ci * _C
            pltpu.sync_copy(src.at[pl.ds(base, _C)], idxs_v)
            pltpu.sync_copy(dst.at[pl.ds(base, _C)], idxd_v)
            cp1 = pltpu.async_copy(tbl_src.at[idxs_v], gsrc_v, s1)
            cp2 = pltpu.async_copy(tbl_dst.at[idxd_v], gdst_v, s2)
            cp3 = pltpu.async_copy(ef.at[pl.ds(base, _C)], ef_v, s3)
            cp1.wait()
            cp2.wait()
            cp3.wait()

            def row(r, c2):
                for k4 in range(4):
                    sl = pl.ds(k4 * 16, 16)
                    sh = pl.ds(64 + k4 * 16, 16)
                    mv = gsrc_v[r, sl] + gdst_v[r, sl] + ef_v[r, sl]
                    sg = 1.0 / (1.0 + jnp.exp(-mv))
                    m_v[r, sl] = mv
                    co_v[r, sl] = gsrc_v[r, sh] * sg
                    co_v[r, sh] = sg
                return c2

            lax.fori_loop(0, _C, row, 0)
            if write_m:
                pltpu.sync_copy(m_v, m_out.at[pl.ds(base, _C)])
            pltpu.sync_copy(co_v, c_out.at[pl.ds(base, _C)])
            return carry

        lax.fori_loop(c0, c1, chunk, 0)

    return k


@functools.lru_cache(None)
def _edge_scatter_call(e, S, P):
    nchunk = e // _C
    npad = 2 * P * S
    rpt = S // 16
    pieces = []
    off = 0
    while off < rpt:
        sz = min(128, rpt - off)
        pieces.append((off, sz))
        off += sz
    mesh = plsc.VectorSubcoreMesh(core_axis_name="c", subcore_axis_name="s")
    scratch = [
        pltpu.VMEM((_C,), jnp.int32), pltpu.VMEM((_C,), jnp.int32),
        pltpu.VMEM((_C, 128), jnp.float32),
        pltpu.VMEM((128, 128), jnp.float32),
        pltpu.VMEM((128, 128), jnp.float32),
        pltpu.VMEM_SHARED((S + 16, 128), jnp.float32),
    ]

    @functools.partial(pl.kernel, mesh=mesh,
                       out_type=jax.ShapeDtypeStruct((npad, 128), jnp.float32),
                       scratch_types=scratch)
    def k(contrib, dstidx, out, idxd_v, idxl_v, co_v, z_v, w_v, acc_sp):
        core = lax.axis_index("c")
        sid = lax.axis_index("s")

        def zrow(r, c2):
            for k4 in range(8):
                z_v[r, pl.ds(k4 * 16, 16)] = jnp.zeros((16,), jnp.float32)
            return c2

        lax.fori_loop(0, 128, zrow, 0)
        c0 = sid * nchunk // 16
        c1 = (sid + 1) * nchunk // 16
        for p in range(P):
            base_seg = pl.multiple_of((2 * p + core) * S, 8)
            row0 = pl.multiple_of(sid * rpt, 8)
            for (o, sz) in pieces:
                pltpu.sync_copy(z_v.at[pl.ds(0, sz)],
                                acc_sp.at[pl.ds(row0 + o, sz)])

            @pl.when(sid == 0)
            def _():
                pltpu.sync_copy(z_v.at[pl.ds(0, 16)], acc_sp.at[pl.ds(S, 16)])

            plsc.subcore_barrier()

            def chunk(ci, carry):
                base = ci * _C
                pltpu.sync_copy(dstidx.at[pl.ds(base, _C)], idxd_v)
                pltpu.sync_copy(contrib.at[pl.ds(base, _C)], co_v)
                for k8 in range(8):
                    sl = pl.ds(k8 * 16, 16)
                    iv = idxd_v[sl] - base_seg
                    ok = (iv >= 0) & (iv < S)
                    idxl_v[sl] = jnp.where(ok, iv, S)
                pltpu.sync_copy(co_v, acc_sp.at[idxl_v], add=True)
                return carry

            lax.fori_loop(c0, c1, chunk, 0)
            plsc.subcore_barrier()
            for (o, sz) in pieces:
                pltpu.sync_copy(acc_sp.at[pl.ds(row0 + o, sz)],
                                w_v.at[pl.ds(0, sz)])
                pltpu.sync_copy(w_v.at[pl.ds(0, sz)],
                                out.at[pl.ds(pl.multiple_of(base_seg + row0 + o, 8), sz)])

    return k


def _egc_fused(p, srci, dsti, pieces_n, pieces_e, stats_of, n_nodes, e_edges,
               S, P, write_m):
    stn = [stats_of(a) for a in pieces_n]
    W1 = jnp.concatenate([p['W_src_gate'], p['W_dst_update']], axis=1)
    W2 = jnp.concatenate([p['W_dst_gate'],
                          jnp.zeros_like(p['W_dst_gate'])], axis=1)
    tbl_src, tbl_dst, xwsu = _mm(pieces_n, stn, p['bn_n_g'], p['bn_n_b'],
                                 [W1, W2, p['W_src_update']])
    ste = [stats_of(a) for a in pieces_e]
    (ye,) = _mm(pieces_e, ste, p['bn_e_g'], p['bn_e_b'], [p['W_edge_gate']])
    ec = _edge_compute_call(e_edges, write_m)
    if write_m:
        m, contrib = ec(tbl_src, tbl_dst, ye, srci, dsti)
    else:
        out = ec(tbl_src, tbl_dst, ye, srci, dsti)
        contrib = out[0] if isinstance(out, (list, tuple)) else out
        m = None
    acc = _edge_scatter_call(e_edges, S, P)(contrib, dsti)
    BR = 2000 if n_nodes % 2000 == 0 else 1000
    x_out = _combine_call(n_nodes, BR, acc.shape[0])(xwsu, acc)
    return x_out, m


def kernel(x, y, z, edge_index, lg_edge_index, params):
    src = edge_index[0].astype(jnp.int32)
    dst = edge_index[1].astype(jnp.int32)
    lsrc = lg_edge_index[0].astype(jnp.int32)
    ldst = lg_edge_index[1].astype(jnp.int32)
    N = x.shape[0]
    E = y.shape[0]
    T = z.shape[0]
    xs, ys, zs = [x], [y], [z]
    stats_cache = {}

    def stats_of(a):
        key = id(a)
        if key not in stats_cache:
            stats_cache[key] = _stats(a)
        return stats_cache[key]

    for i in range(_L):
        lp = params['layers'][i]
        nx, ny = _egc_fused(lp['node_update'], src, dst, xs, ys, stats_of,
                            N, E, S=5120, P=1, write_m=True)
        ny2, nz = _egc_fused(lp['edge_update'], lsrc, ldst, [ny], zs,
                             stats_of, E, T, S=9984, P=9,
                             write_m=(i < _L - 1))
        xs.append(nx)
        ys.append(ny2)
        if nz is not None:
            zs.append(nz)

    bx = params['bottleneck_x']
    by = params['bottleneck_y']
    (x_out,) = _mm(xs, [stats_of(a) for a in xs], bx['g'], bx['b'],
                   [bx['W']], residual=x)
    (y_out,) = _mm(ys, [stats_of(a) for a in ys], by['g'], by['b'],
                   [by['W']], residual=y)
    return (x_out, y_out)


# speedup vs baseline: 1.7240x; 1.1562x over previous
import functools

import jax
import jax.numpy as jnp
from jax import lax
from jax.experimental import pallas as pl
from jax.experimental.pallas import tpu as pltpu
from jax.experimental.pallas import tpu_sc as plsc

_EPS_BN = 1e-5
_EPS_SEG = 1e-6
_C = 128
_NW = 32
_L = 3


@functools.lru_cache(None)
def _stats_call(R, F, BR):
    def body(a_ref, o_ref):
        a = a_ref[...]

        @pl.when(pl.program_id(0) == 0)
        def _():
            o_ref[...] = jnp.zeros_like(o_ref)

        o_ref[0:1, :] += jnp.sum(a, axis=0, keepdims=True)
        o_ref[1:2, :] += jnp.sum(a * a, axis=0, keepdims=True)

    return pl.pallas_call(
        body,
        grid=(R // BR,),
        in_specs=[pl.BlockSpec((BR, F), lambda i: (i, 0))],
        out_specs=pl.BlockSpec((8, F), lambda i: (0, 0)),
        out_shape=jax.ShapeDtypeStruct((8, F), jnp.float32),
    )


def _stats(a):
    R, F = a.shape
    BR = 2000 if R % 2000 == 0 else 1000
    return _stats_call(R, F, BR)(a)


@functools.lru_cache(None)
def _mm_call(R, BR, npieces, Ks, residual):
    nouts = len(Ks)

    def body(*refs):
        it = iter(refs)
        a = [next(it) for _ in range(npieces)]
        st = [next(it) for _ in range(npieces)]
        g = [next(it) for _ in range(npieces)]
        b = [next(it) for _ in range(npieces)]
        W = [[next(it) for _ in range(npieces)] for _ in range(nouts)]
        res = next(it) if residual else None
        outs = [next(it) for _ in range(nouts)]
        acc = [None] * nouts
        inv_r = 1.0 / R
        for j in range(npieces):
            aj = a[j][...]
            mean = st[j][0:1, :] * inv_r
            var = st[j][1:2, :] * inv_r - mean * mean
            xn = (aj - mean) * lax.rsqrt(var + _EPS_BN) * g[j][...] + b[j][...]
            xn = xn * jax.nn.sigmoid(xn)
            for o in range(nouts):
                d = jnp.dot(xn, W[o][j][...], preferred_element_type=jnp.float32)
                acc[o] = d if acc[o] is None else acc[o] + d
        for o in range(nouts):
            val = acc[o]
            if residual and o == 0:
                val = val + res[...]
            outs[o][...] = val

    in_specs = (
        [pl.BlockSpec((BR, 64), lambda i: (i, 0)) for _ in range(npieces)]
        + [pl.BlockSpec((8, 64), lambda i: (0, 0)) for _ in range(npieces)]
        + [pl.BlockSpec((1, 64), lambda i: (0, 0)) for _ in range(2 * npieces)]
        + [pl.BlockSpec((64, K), lambda i: (0, 0))
           for K in Ks for _ in range(npieces)]
        + ([pl.BlockSpec((BR, Ks[0]), lambda i: (i, 0))] if residual else [])
    )
    return pl.pallas_call(
        body,
        grid=(R // BR,),
        in_specs=in_specs,
        out_specs=[pl.BlockSpec((BR, K), lambda i: (i, 0)) for K in Ks],
        out_shape=[jax.ShapeDtypeStruct((R, K), jnp.float32) for K in Ks],
    )


def _mm(pieces, stats, g, b, Ws, residual=None):
    R = pieces[0].shape[0]
    npieces = len(pieces)
    Ks = tuple(int(W.shape[1]) for W in Ws)
    BR = 2000 if R % 2000 == 0 else 1000
    args = list(pieces) + list(stats)
    for j in range(npieces):
        args.append(g[64 * j:64 * (j + 1)].reshape(1, 64))
    for j in range(npieces):
        args.append(b[64 * j:64 * (j + 1)].reshape(1, 64))
    for W in Ws:
        for j in range(npieces):
            args.append(W[64 * j:64 * (j + 1), :])
    if residual is not None:
        args.append(residual)
    out = _mm_call(R, BR, npieces, Ks, residual is not None)(*args)
    return out


@functools.lru_cache(None)
def _combine_call(R, BR, Racc):
    def body(su_ref, acc_ref, o_ref):
        accv = acc_ref[...]
        o_ref[...] = su_ref[...] + accv[:, 0:64] / (accv[:, 64:128] + _EPS_SEG)

    return pl.pallas_call(
        body,
        grid=(R // BR,),
        in_specs=[pl.BlockSpec((BR, 64), lambda i: (i, 0)),
                  pl.BlockSpec((BR, 128), lambda i: (i, 0))],
        out_specs=pl.BlockSpec((BR, 64), lambda i: (i, 0)),
        out_shape=jax.ShapeDtypeStruct((R, 64), jnp.float32),
    )


@functools.lru_cache(None)
def _edge_compute_call(e, write_m):
    nchunk = e // _C
    mesh = plsc.VectorSubcoreMesh(core_axis_name="c", subcore_axis_name="s")
    out_type = [jax.ShapeDtypeStruct((e, 128), jnp.float32)]
    if write_m:
        out_type = [jax.ShapeDtypeStruct((e, 64), jnp.float32)] + out_type
    scratch = [
        pltpu.VMEM((_C,), jnp.int32), pltpu.VMEM((_C,), jnp.int32),
        pltpu.VMEM((_C,), jnp.int32), pltpu.VMEM((_C,), jnp.int32),
        pltpu.VMEM((_C, 128), jnp.float32), pltpu.VMEM((_C, 128), jnp.float32),
        pltpu.VMEM((_C, 128), jnp.float32), pltpu.VMEM((_C, 128), jnp.float32),
        pltpu.VMEM((_C, 64), jnp.float32), pltpu.VMEM((_C, 64), jnp.float32),
        pltpu.VMEM((_C, 128), jnp.float32),
        pltpu.SemaphoreType.DMA, pltpu.SemaphoreType.DMA,
    ]

    @functools.partial(pl.kernel, mesh=mesh, out_type=out_type,
                       scratch_types=scratch)
    def k(tbl_src, tbl_dst, ef, src, dst, *rest):
        ri = iter(rest)
        m_out = next(ri) if write_m else None
        c_out = next(ri)
        idxs_v = [next(ri), next(ri)]
        idxd_v = [next(ri), next(ri)]
        gsrc_v = [next(ri), next(ri)]
        gdst_v = [next(ri), next(ri)]
        ef_v = next(ri)
        m_v = next(ri)
        co_v = next(ri)
        gsem = [next(ri), next(ri)]
        wid = lax.axis_index("s") * 2 + lax.axis_index("c")
        c0 = wid * nchunk // _NW
        c1 = (wid + 1) * nchunk // _NW

        def start(ci, b):
            base = ci * _C
            pltpu.sync_copy(src.at[pl.ds(base, _C)], idxs_v[b])
            pltpu.sync_copy(dst.at[pl.ds(base, _C)], idxd_v[b])
            pltpu.async_copy(tbl_src.at[idxs_v[b]], gsrc_v[b], gsem[b])
            pltpu.async_copy(tbl_dst.at[idxd_v[b]], gdst_v[b], gsem[b])

        start(c0, 0)

        def chunk(ci, carry):
            par = lax.rem(ci - c0, 2)
            for b in (0, 1):
                @pl.when(par == b)
                def _():
                    @pl.when(ci + 1 < c1)
                    def _():
                        start(ci + 1, 1 - b)
                    pltpu.make_async_copy(
                        tbl_src.at[idxs_v[b]], gsrc_v[b], gsem[b]).wait()
                    pltpu.make_async_copy(
                        tbl_dst.at[idxd_v[b]], gdst_v[b], gsem[b]).wait()
                    pltpu.sync_copy(ef.at[pl.ds(ci * _C, _C)], ef_v)

                    def row(r, c2):
                        for k4 in range(4):
                            sl = pl.ds(k4 * 16, 16)
                            sh = pl.ds(64 + k4 * 16, 16)
                            mv = (gsrc_v[b][r, sl] + gdst_v[b][r, sl]
                                  + ef_v[r, sl])
                            sg = 1.0 / (1.0 + jnp.exp(-mv))
                            m_v[r, sl] = mv
                            co_v[r, sl] = gsrc_v[b][r, sh] * sg
                            co_v[r, sh] = sg
                        return c2

                    lax.fori_loop(0, _C, row, 0)
                    base = ci * _C
                    if write_m:
                        pltpu.sync_copy(m_v, m_out.at[pl.ds(base, _C)])
                    pltpu.sync_copy(co_v, c_out.at[pl.ds(base, _C)])
            return carry

        lax.fori_loop(c0, c1, chunk, 0)

    return k


@functools.lru_cache(None)
def _edge_scatter_call(e, S, P):
    nchunk = e // _C
    npad = 2 * P * S
    rpt = S // 16
    pieces = []
    off = 0
    while off < rpt:
        sz = min(64, rpt - off)
        pieces.append((off, sz))
        off += sz
    mesh = plsc.VectorSubcoreMesh(core_axis_name="c", subcore_axis_name="s")
    scratch = [
        pltpu.VMEM((_C,), jnp.int32), pltpu.VMEM((_C,), jnp.int32),
        pltpu.VMEM((_C,), jnp.int32), pltpu.VMEM((_C,), jnp.int32),
        pltpu.VMEM((_C, 128), jnp.float32), pltpu.VMEM((_C, 128), jnp.float32),
        pltpu.VMEM((64, 128), jnp.float32),
        pltpu.VMEM((64, 128), jnp.float32),
        pltpu.VMEM_SHARED((S + 16, 128), jnp.float32),
        pltpu.SemaphoreType.DMA, pltpu.SemaphoreType.DMA,
    ]

    @functools.partial(pl.kernel, mesh=mesh,
                       out_type=jax.ShapeDtypeStruct((npad, 128), jnp.float32),
                       scratch_types=scratch)
    def k(contrib, dstidx, out, i0, i1, l0, l1, cv0, cv1, z_v, w_v, acc_sp,
          sm0, sm1):
        idxd_v = [i0, i1]
        idxl_v = [l0, l1]
        co_v = [cv0, cv1]
        lsem = [sm0, sm1]
        core = lax.axis_index("c")
        sid = lax.axis_index("s")

        def start(ci, b):
            base = ci * _C
            pltpu.async_copy(dstidx.at[pl.ds(base, _C)], idxd_v[b], lsem[b])
            pltpu.async_copy(contrib.at[pl.ds(base, _C)], co_v[b], lsem[b])

        def zrow(r, c2):
            for k4 in range(8):
                z_v[r, pl.ds(k4 * 16, 16)] = jnp.zeros((16,), jnp.float32)
            return c2

        lax.fori_loop(0, 64, zrow, 0)
        c0 = sid * nchunk // 16
        c1 = (sid + 1) * nchunk // 16
        for p in range(P):
            base_seg = pl.multiple_of((2 * p + core) * S, 8)
            row0 = pl.multiple_of(sid * rpt, 8)
            for (o, sz) in pieces:
                pltpu.sync_copy(z_v.at[pl.ds(0, sz)],
                                acc_sp.at[pl.ds(row0 + o, sz)])

            @pl.when(sid == 0)
            def _():
                pltpu.sync_copy(z_v.at[pl.ds(0, 16)], acc_sp.at[pl.ds(S, 16)])

            plsc.subcore_barrier()
            start(c0, 0)

            def chunk(ci, carry):
                par = lax.rem(ci - c0, 2)
                for b in (0, 1):
                    @pl.when(par == b)
                    def _():
                        @pl.when(ci + 1 < c1)
                        def _():
                            start(ci + 1, 1 - b)
                        pltpu.make_async_copy(
                            dstidx.at[pl.ds(0, _C)], idxd_v[b],
                            lsem[b]).wait()
                        pltpu.make_async_copy(
                            contrib.at[pl.ds(0, _C)], co_v[b],
                            lsem[b]).wait()
                        for k8 in range(8):
                            sl = pl.ds(k8 * 16, 16)
                            iv = idxd_v[b][sl] - base_seg
                            ok = (iv >= 0) & (iv < S)
                            idxl_v[b][sl] = jnp.where(ok, iv, S)
                        pltpu.sync_copy(co_v[b], acc_sp.at[idxl_v[b]],
                                        add=True)
                return carry

            lax.fori_loop(c0, c1, chunk, 0)
            plsc.subcore_barrier()
            for (o, sz) in pieces:
                pltpu.sync_copy(acc_sp.at[pl.ds(row0 + o, sz)],
                                w_v.at[pl.ds(0, sz)])
                pltpu.sync_copy(w_v.at[pl.ds(0, sz)],
                                out.at[pl.ds(pl.multiple_of(base_seg + row0 + o, 8), sz)])

    return k


def _egc_fused(p, srci, dsti, pieces_n, pieces_e, stats_of, n_nodes, e_edges,
               S, P, write_m):
    stn = [stats_of(a) for a in pieces_n]
    W1 = jnp.concatenate([p['W_src_gate'], p['W_dst_update']], axis=1)
    W2 = jnp.concatenate([p['W_dst_gate'],
                          jnp.zeros_like(p['W_dst_gate'])], axis=1)
    tbl_src, tbl_dst, xwsu = _mm(pieces_n, stn, p['bn_n_g'], p['bn_n_b'],
                                 [W1, W2, p['W_src_update']])
    ste = [stats_of(a) for a in pieces_e]
    (ye,) = _mm(pieces_e, ste, p['bn_e_g'], p['bn_e_b'], [p['W_edge_gate']])
    ec = _edge_compute_call(e_edges, write_m)
    if write_m:
        m, contrib = ec(tbl_src, tbl_dst, ye, srci, dsti)
    else:
        out = ec(tbl_src, tbl_dst, ye, srci, dsti)
        contrib = out[0] if isinstance(out, (list, tuple)) else out
        m = None
    acc = _edge_scatter_call(e_edges, S, P)(contrib, dsti)
    BR = 2000 if n_nodes % 2000 == 0 else 1000
    x_out = _combine_call(n_nodes, BR, acc.shape[0])(xwsu, acc)
    return x_out, m


def kernel(x, y, z, edge_index, lg_edge_index, params):
    src = edge_index[0].astype(jnp.int32)
    dst = edge_index[1].astype(jnp.int32)
    lsrc = lg_edge_index[0].astype(jnp.int32)
    ldst = lg_edge_index[1].astype(jnp.int32)
    N = x.shape[0]
    E = y.shape[0]
    T = z.shape[0]
    xs, ys, zs = [x], [y], [z]
    stats_cache = {}

    def stats_of(a):
        key = id(a)
        if key not in stats_cache:
            stats_cache[key] = _stats(a)
        return stats_cache[key]

    for i in range(_L):
        lp = params['layers'][i]
        nx, ny = _egc_fused(lp['node_update'], src, dst, xs, ys, stats_of,
                            N, E, S=5120, P=1, write_m=True)
        ny2, nz = _egc_fused(lp['edge_update'], lsrc, ldst, [ny], zs,
                             stats_of, E, T, S=10112, P=8,
                             write_m=(i < _L - 1))
        xs.append(nx)
        ys.append(ny2)
        if nz is not None:
            zs.append(nz)

    bx = params['bottleneck_x']
    by = params['bottleneck_y']
    (x_out,) = _mm(xs, [stats_of(a) for a in xs], bx['g'], bx['b'],
                   [bx['W']], residual=x)
    (y_out,) = _mm(ys, [stats_of(a) for a in ys], by['g'], by['b'],
                   [by['W']], residual=y)
    return (x_out, y_out)


# speedup vs baseline: 1.9910x; 1.1549x over previous
import functools

import jax
import jax.numpy as jnp
from jax import lax
from jax.experimental import pallas as pl
from jax.experimental.pallas import tpu as pltpu
from jax.experimental.pallas import tpu_sc as plsc

_EPS_BN = 1e-5
_EPS_SEG = 1e-6
_C = 128
_NW = 32
_L = 3


@functools.lru_cache(None)
def _stats_call(R, F, BR):
    def body(a_ref, o_ref):
        a = a_ref[...]

        @pl.when(pl.program_id(0) == 0)
        def _():
            o_ref[...] = jnp.zeros_like(o_ref)

        o_ref[0:1, :] += jnp.sum(a, axis=0, keepdims=True)
        o_ref[1:2, :] += jnp.sum(a * a, axis=0, keepdims=True)

    return pl.pallas_call(
        body,
        grid=(R // BR,),
        in_specs=[pl.BlockSpec((BR, F), lambda i: (i, 0))],
        out_specs=pl.BlockSpec((8, F), lambda i: (0, 0)),
        out_shape=jax.ShapeDtypeStruct((8, F), jnp.float32),
    )


def _stats(a):
    R, F = a.shape
    BR = 2000 if R % 2000 == 0 else 1000
    return _stats_call(R, F, BR)(a)


@functools.lru_cache(None)
def _mm_call(R, BR, npieces, Ks, residual):
    nouts = len(Ks)

    def body(*refs):
        it = iter(refs)
        a = [next(it) for _ in range(npieces)]
        st = [next(it) for _ in range(npieces)]
        g = [next(it) for _ in range(npieces)]
        b = [next(it) for _ in range(npieces)]
        W = [[next(it) for _ in range(npieces)] for _ in range(nouts)]
        res = next(it) if residual else None
        outs = [next(it) for _ in range(nouts)]
        acc = [None] * nouts
        inv_r = 1.0 / R
        for j in range(npieces):
            aj = a[j][...]
            mean = st[j][0:1, :] * inv_r
            var = st[j][1:2, :] * inv_r - mean * mean
            xn = (aj - mean) * lax.rsqrt(var + _EPS_BN) * g[j][...] + b[j][...]
            xn = xn * jax.nn.sigmoid(xn)
            for o in range(nouts):
                d = jnp.dot(xn, W[o][j][...], preferred_element_type=jnp.float32)
                acc[o] = d if acc[o] is None else acc[o] + d
        for o in range(nouts):
            val = acc[o]
            if residual and o == 0:
                val = val + res[...]
            outs[o][...] = val

    in_specs = (
        [pl.BlockSpec((BR, 64), lambda i: (i, 0)) for _ in range(npieces)]
        + [pl.BlockSpec((8, 64), lambda i: (0, 0)) for _ in range(npieces)]
        + [pl.BlockSpec((1, 64), lambda i: (0, 0)) for _ in range(2 * npieces)]
        + [pl.BlockSpec((64, K), lambda i: (0, 0))
           for K in Ks for _ in range(npieces)]
        + ([pl.BlockSpec((BR, Ks[0]), lambda i: (i, 0))] if residual else [])
    )
    return pl.pallas_call(
        body,
        grid=(R // BR,),
        in_specs=in_specs,
        out_specs=[pl.BlockSpec((BR, K), lambda i: (i, 0)) for K in Ks],
        out_shape=[jax.ShapeDtypeStruct((R, K), jnp.float32) for K in Ks],
    )


def _mm(pieces, stats, g, b, Ws, residual=None):
    R = pieces[0].shape[0]
    npieces = len(pieces)
    Ks = tuple(int(W.shape[1]) for W in Ws)
    BR = 2000 if R % 2000 == 0 else 1000
    args = list(pieces) + list(stats)
    for j in range(npieces):
        args.append(g[64 * j:64 * (j + 1)].reshape(1, 64))
    for j in range(npieces):
        args.append(b[64 * j:64 * (j + 1)].reshape(1, 64))
    for W in Ws:
        for j in range(npieces):
            args.append(W[64 * j:64 * (j + 1), :])
    if residual is not None:
        args.append(residual)
    out = _mm_call(R, BR, npieces, Ks, residual is not None)(*args)
    return out


@functools.lru_cache(None)
def _combine_call(R, BR, Racc):
    def body(su_ref, acc_ref, o_ref):
        accv = acc_ref[...]
        o_ref[...] = su_ref[...] + accv[:, 0:64] / (accv[:, 64:128] + _EPS_SEG)

    return pl.pallas_call(
        body,
        grid=(R // BR,),
        in_specs=[pl.BlockSpec((BR, 64), lambda i: (i, 0)),
                  pl.BlockSpec((BR, 128), lambda i: (i, 0))],
        out_specs=pl.BlockSpec((BR, 64), lambda i: (i, 0)),
        out_shape=jax.ShapeDtypeStruct((R, 64), jnp.float32),
    )


@functools.lru_cache(None)
def _edge_compute_call(e, write_m):
    nchunk = e // _C
    mesh = plsc.VectorSubcoreMesh(core_axis_name="c", subcore_axis_name="s")
    out_type = [jax.ShapeDtypeStruct((e, 128), jnp.float32)]
    if write_m:
        out_type = [jax.ShapeDtypeStruct((e, 64), jnp.float32)] + out_type
    scratch = [
        pltpu.VMEM((_C,), jnp.int32), pltpu.VMEM((_C,), jnp.int32),
        pltpu.VMEM((_C,), jnp.int32), pltpu.VMEM((_C,), jnp.int32),
        pltpu.VMEM((_C, 128), jnp.float32), pltpu.VMEM((_C, 128), jnp.float32),
        pltpu.VMEM((_C, 128), jnp.float32), pltpu.VMEM((_C, 128), jnp.float32),
        pltpu.VMEM((_C, 64), jnp.float32), pltpu.VMEM((_C, 64), jnp.float32),
        pltpu.VMEM((_C, 128), jnp.float32),
        pltpu.SemaphoreType.DMA, pltpu.SemaphoreType.DMA,
    ]

    @functools.partial(pl.kernel, mesh=mesh, out_type=out_type,
                       scratch_types=scratch)
    def k(tbl_src, tbl_dst, ef, src, dst, *rest):
        ri = iter(rest)
        m_out = next(ri) if write_m else None
        c_out = next(ri)
        idxs_v = [next(ri), next(ri)]
        idxd_v = [next(ri), next(ri)]
        gsrc_v = [next(ri), next(ri)]
        gdst_v = [next(ri), next(ri)]
        ef_v = next(ri)
        m_v = next(ri)
        co_v = next(ri)
        gsem = [next(ri), next(ri)]
        wid = lax.axis_index("s") * 2 + lax.axis_index("c")
        c0 = wid * nchunk // _NW
        c1 = (wid + 1) * nchunk // _NW

        def start(ci, b):
            base = ci * _C
            pltpu.sync_copy(src.at[pl.ds(base, _C)], idxs_v[b])
            pltpu.sync_copy(dst.at[pl.ds(base, _C)], idxd_v[b])
            pltpu.async_copy(tbl_src.at[idxs_v[b]], gsrc_v[b], gsem[b])
            pltpu.async_copy(tbl_dst.at[idxd_v[b]], gdst_v[b], gsem[b])

        start(c0, 0)

        def chunk(ci, carry):
            par = lax.rem(ci - c0, 2)
            for b in (0, 1):
                @pl.when(par == b)
                def _():
                    @pl.when(ci + 1 < c1)
                    def _():
                        start(ci + 1, 1 - b)
                    pltpu.make_async_copy(
                        tbl_src.at[idxs_v[b]], gsrc_v[b], gsem[b]).wait()
                    pltpu.make_async_copy(
                        tbl_dst.at[idxd_v[b]], gdst_v[b], gsem[b]).wait()
                    pltpu.sync_copy(ef.at[pl.ds(ci * _C, _C)], ef_v)

                    def row(r, c2):
                        for k4 in range(4):
                            sl = pl.ds(k4 * 16, 16)
                            sh = pl.ds(64 + k4 * 16, 16)
                            mv = (gsrc_v[b][r, sl] + gdst_v[b][r, sl]
                                  + ef_v[r, sl])
                            sg = 1.0 / (1.0 + jnp.exp(-mv))
                            m_v[r, sl] = mv
                            co_v[r, sl] = gsrc_v[b][r, sh] * sg
                            co_v[r, sh] = sg
                        return c2

                    lax.fori_loop(0, _C, row, 0)
                    base = ci * _C
                    if write_m:
                        pltpu.sync_copy(m_v, m_out.at[pl.ds(base, _C)])
                    pltpu.sync_copy(co_v, c_out.at[pl.ds(base, _C)])
            return carry

        lax.fori_loop(c0, c1, chunk, 0)

    return k


_CS = 32


@functools.lru_cache(None)
def _edge_scatter_call(e, S, P):
    nchunk = e // _CS
    npad = 2 * P * S
    rpt = S // 16
    pieces = []
    off = 0
    while off < rpt:
        sz = min(16, rpt - off)
        pieces.append((off, sz))
        off += sz
    mesh = plsc.VectorSubcoreMesh(core_axis_name="c", subcore_axis_name="s")
    scratch = [
        pltpu.VMEM((_CS,), jnp.int32), pltpu.VMEM((_CS,), jnp.int32),
        pltpu.VMEM((_CS,), jnp.int32), pltpu.VMEM((_CS,), jnp.int32),
        pltpu.VMEM((_CS, 128), jnp.float32),
        pltpu.VMEM((_CS, 128), jnp.float32),
        pltpu.VMEM((16, 128), jnp.float32),
        pltpu.VMEM((16, 128), jnp.float32),
        pltpu.VMEM_SHARED((S + 16, 128), jnp.float32),
        pltpu.SemaphoreType.DMA, pltpu.SemaphoreType.DMA,
    ]

    @functools.partial(pl.kernel, mesh=mesh,
                       out_type=jax.ShapeDtypeStruct((npad, 128), jnp.float32),
                       scratch_types=scratch)
    def k(contrib, dstidx, out, i0, i1, l0, l1, cv0, cv1, z_v, w_v, acc_sp,
          sm0, sm1):
        idxd_v = [i0, i1]
        idxl_v = [l0, l1]
        co_v = [cv0, cv1]
        lsem = [sm0, sm1]
        core = lax.axis_index("c")
        sid = lax.axis_index("s")

        def start(ci, b):
            base = ci * _CS
            pltpu.async_copy(dstidx.at[pl.ds(base, _CS)], idxd_v[b], lsem[b])
            pltpu.async_copy(contrib.at[pl.ds(base, _CS)], co_v[b], lsem[b])

        def zrow(r, c2):
            for k4 in range(8):
                z_v[r, pl.ds(k4 * 16, 16)] = jnp.zeros((16,), jnp.float32)
            return c2

        lax.fori_loop(0, 16, zrow, 0)
        c0 = sid * nchunk // 16
        c1 = (sid + 1) * nchunk // 16
        for p in range(P):
            base_seg = pl.multiple_of((2 * p + core) * S, 8)
            row0 = pl.multiple_of(sid * rpt, 8)
            for (o, sz) in pieces:
                pltpu.sync_copy(z_v.at[pl.ds(0, sz)],
                                acc_sp.at[pl.ds(row0 + o, sz)])

            @pl.when(sid == 0)
            def _():
                pltpu.sync_copy(z_v.at[pl.ds(0, 16)], acc_sp.at[pl.ds(S, 16)])

            plsc.subcore_barrier()
            start(c0, 0)

            def chunk(ci, carry):
                par = lax.rem(ci - c0, 2)
                for b in (0, 1):
                    @pl.when(par == b)
                    def _():
                        @pl.when(ci + 1 < c1)
                        def _():
                            start(ci + 1, 1 - b)
                        pltpu.make_async_copy(
                            dstidx.at[pl.ds(0, _CS)], idxd_v[b],
                            lsem[b]).wait()
                        pltpu.make_async_copy(
                            contrib.at[pl.ds(0, _CS)], co_v[b],
                            lsem[b]).wait()
                        for k8 in range(_CS // 16):
                            sl = pl.ds(k8 * 16, 16)
                            iv = idxd_v[b][sl] - base_seg
                            ok = (iv >= 0) & (iv < S)
                            idxl_v[b][sl] = jnp.where(ok, iv, S)
                        pltpu.sync_copy(co_v[b], acc_sp.at[idxl_v[b]],
                                        add=True)
                return carry

            lax.fori_loop(c0, c1, chunk, 0)
            plsc.subcore_barrier()
            for (o, sz) in pieces:
                pltpu.sync_copy(acc_sp.at[pl.ds(row0 + o, sz)],
                                w_v.at[pl.ds(0, sz)])
                pltpu.sync_copy(w_v.at[pl.ds(0, sz)],
                                out.at[pl.ds(pl.multiple_of(base_seg + row0 + o, 8), sz)])

    return k


_BS = 8192


def _take16(v, idx):
    dnums = lax.GatherDimensionNumbers(
        offset_dims=(), collapsed_slice_dims=(0,), start_index_map=(0,))
    return lax.gather(v, idx[:, None], dnums, slice_sizes=(1,),
                      mode=lax.GatherScatterMode.PROMISE_IN_BOUNDS)
_CAP = 1024


@functools.lru_cache(None)
def _partition_call(e, nbuk):
    nchunk = e // _C
    rows = _NW * nbuk * _CAP + 8
    trash = rows - 8
    big = 1 << 28
    mesh = plsc.VectorSubcoreMesh(core_axis_name="c", subcore_axis_name="s")
    out_type = [
        jax.ShapeDtypeStruct((rows, 128), jnp.float32),
        jax.ShapeDtypeStruct((_NW, nbuk * _CAP), jnp.int32),
        jax.ShapeDtypeStruct((_NW, 128), jnp.int32),
    ]
    scratch = [
        pltpu.VMEM((_C,), jnp.int32), pltpu.VMEM((_C,), jnp.int32),
        pltpu.VMEM((_C, 128), jnp.float32), pltpu.VMEM((_C, 128), jnp.float32),
        pltpu.VMEM((_C,), jnp.int32),
        pltpu.VMEM((nbuk * _CAP,), jnp.int32),
        pltpu.VMEM((128,), jnp.int32),
        pltpu.VMEM((128,), jnp.int32),
        pltpu.SemaphoreType.DMA, pltpu.SemaphoreType.DMA,
    ]

    @functools.partial(pl.kernel, mesh=mesh, out_type=out_type,
                       scratch_types=scratch)
    def k(contrib, dstidx, rout, bout, cout, i0, i1, cv0, cv1, gidx_v,
          stage_v, cflush_v, cnt_v, sm0, sm1):
        idxd_v = [i0, i1]
        co_v = [cv0, cv1]
        lsem = [sm0, sm1]
        wid = lax.axis_index("s") * 2 + lax.axis_index("c")
        c0 = wid * nchunk // _NW
        c1 = (wid + 1) * nchunk // _NW

        ones16 = jnp.ones((16,), jnp.int32)
        zeros16 = jnp.zeros((16,), jnp.int32)
        lanes = lax.iota(jnp.int32, 16)
        for j in range(8):
            cnt_v[pl.ds(j * 16, 16)] = zeros16
        for j in range(8):
            cflush_v[pl.ds(j * 16, 16)] = zeros16

        def initstage(j, carry):
            stage_v[pl.ds(j * 16, 16)] = zeros16 + big
            return carry

        lax.fori_loop(0, nbuk * _CAP // 16, initstage, 0)

        def start(ci, b):
            base = ci * _C
            pltpu.async_copy(dstidx.at[pl.ds(base, _C)], idxd_v[b], lsem[b])
            pltpu.async_copy(contrib.at[pl.ds(base, _C)], co_v[b], lsem[b])

        start(c0, 0)

        def chunk(ci, carry):
            par = lax.rem(ci - c0, 2)
            for b in (0, 1):
                @pl.when(par == b)
                def _():
                    @pl.when(ci + 1 < c1)
                    def _():
                        start(ci + 1, 1 - b)
                    pltpu.make_async_copy(
                        dstidx.at[pl.ds(0, _C)], idxd_v[b], lsem[b]).wait()
                    pltpu.make_async_copy(
                        contrib.at[pl.ds(0, _C)], co_v[b], lsem[b]).wait()
                    for k8 in range(8):
                        sl = pl.ds(k8 * 16, 16)
                        iv = idxd_v[b][sl]
                        bk = lax.shift_right_logical(iv, 13)
                        rank = zeros16
                        after = zeros16
                        for d in range(1, 16):
                            dn = _take16(bk, jnp.maximum(lanes - d, 0))
                            up = _take16(bk, jnp.minimum(lanes + d, 15))
                            rank = rank + jnp.where(
                                (dn == bk) & (lanes >= d), ones16, zeros16)
                            after = after + jnp.where(
                                (up == bk) & (lanes < 16 - d), ones16,
                                zeros16)
                        base_c = plsc.load_gather(cnt_v, [bk])
                        slot = base_c + rank
                        okc = slot < _CAP
                        plsc.store_scatter(cnt_v, [bk], slot + 1,
                                           mask=(after == 0))
                        g = wid * nbuk * _CAP + bk * _CAP + slot
                        gidx_v[sl] = jnp.where(okc, g, zeros16 + trash)
                        plsc.store_scatter(stage_v, [bk * _CAP + slot], iv,
                                           mask=okc)
                    pltpu.sync_copy(co_v[b], rout.at[gidx_v])
                return carry

        lax.fori_loop(c0, c1, chunk, 0)
        cflush_v[pl.ds(0, 16)] = cnt_v[pl.ds(0, 16)]
        cflush_v[pl.ds(16, 16)] = cnt_v[pl.ds(16, 16)]
        pltpu.sync_copy(stage_v, bout.at[wid])
        pltpu.sync_copy(cflush_v, cout.at[wid])

    return k


@functools.lru_cache(None)
def _bucket_acc_call(nbuk):
    sacc = _BS + 128
    rpt_z = sacc // 16
    rpt_w = _BS // 16
    mesh = plsc.VectorSubcoreMesh(core_axis_name="c", subcore_axis_name="s")
    scratch = [
        pltpu.VMEM((_C,), jnp.int32), pltpu.VMEM((_C,), jnp.int32),
        pltpu.VMEM((_C,), jnp.int32), pltpu.VMEM((_C,), jnp.int32),
        pltpu.VMEM((_C, 128), jnp.float32), pltpu.VMEM((_C, 128), jnp.float32),
        pltpu.VMEM((64, 128), jnp.float32),
        pltpu.VMEM((64, 128), jnp.float32),
        pltpu.VMEM((_NW, 128), jnp.int32),
        pltpu.VMEM_SHARED((sacc, 128), jnp.float32),
        pltpu.SemaphoreType.DMA, pltpu.SemaphoreType.DMA,
    ]

    @functools.partial(
        pl.kernel, mesh=mesh,
        out_type=jax.ShapeDtypeStruct((nbuk * _BS, 128), jnp.float32),
        scratch_types=scratch)
    def k(rowsbuf, bidx, counts, out, i0, i1, l0, l1, cv0, cv1, z_v, w_v,
          cnt_v, acc_sp, sm0, sm1):
        idxd_v = [i0, i1]
        idxl_v = [l0, l1]
        co_v = [cv0, cv1]
        lsem = [sm0, sm1]
        core = lax.axis_index("c")
        sid = lax.axis_index("s")
        pltpu.sync_copy(counts, cnt_v)

        def zrow(r, c2):
            for k4 in range(8):
                z_v[r, pl.ds(k4 * 16, 16)] = jnp.zeros((16,), jnp.float32)
            return c2

        lax.fori_loop(0, 64, zrow, 0)

        def start(regbase, j, b):
            base = regbase + j * _C
            pltpu.async_copy(rowsbuf.at[pl.ds(base, _C)], co_v[b], lsem[b])

        def startidx(pt, bu, j, b):
            pltpu.async_copy(
                bidx.at[pt, pl.ds(bu * _CAP + j * _C, _C)], idxd_v[b],
                lsem[b])

        for bu in range(nbuk):
            @pl.when(core == (bu % 2))
            def _():
                row0 = pl.multiple_of(sid * rpt_z, 8)
                for o in range(0, rpt_z, 64):
                    sz = min(64, rpt_z - o)
                    pltpu.sync_copy(z_v.at[pl.ds(0, sz)],
                                    acc_sp.at[pl.ds(row0 + o, sz)])
                plsc.subcore_barrier()
                for pt_off in (0, 16):
                    pt = sid + pt_off
                    cvec = cnt_v[pt, pl.ds((bu // 16) * 16, 16)]
                    cnt = cvec[bu % 16]
                    nch = jnp.minimum(
                        lax.shift_right_logical(cnt + (_C - 1), 7),
                        _CAP // _C)
                    regbase = (pt * nbuk + bu) * _CAP

                    @pl.when(nch > 0)
                    def _():
                        startidx(pt, bu, 0, 0)
                        start(regbase, 0, 0)

                        def chunk(j, carry):
                            par = lax.rem(j, 2)
                            for b in (0, 1):
                                @pl.when(par == b)
                                def _():
                                    @pl.when(j + 1 < nch)
                                    def _():
                                        startidx(pt, bu, j + 1, 1 - b)
                                        start(regbase, j + 1, 1 - b)
                                    pltpu.make_async_copy(
                                        bidx.at[0, pl.ds(0, _C)],
                                        idxd_v[b], lsem[b]).wait()
                                    pltpu.make_async_copy(
                                        rowsbuf.at[pl.ds(0, _C)],
                                        co_v[b], lsem[b]).wait()
                                    for k8 in range(8):
                                        sl = pl.ds(k8 * 16, 16)
                                        iv = idxd_v[b][sl] - bu * _BS
                                        ok = (iv >= 0) & (iv < _BS)
                                        idxl_v[b][sl] = jnp.where(
                                            ok, iv, _BS)
                                    pltpu.sync_copy(
                                        co_v[b], acc_sp.at[idxl_v[b]],
                                        add=True)
                            return carry

                        lax.fori_loop(0, nch, chunk, 0)
                plsc.subcore_barrier()
                wrow = pl.multiple_of(sid * rpt_w, 8)
                for o in range(0, rpt_w, 64):
                    sz = min(64, rpt_w - o)
                    pltpu.sync_copy(acc_sp.at[pl.ds(wrow + o, sz)],
                                    w_v.at[pl.ds(0, sz)])
                    pltpu.sync_copy(
                        w_v.at[pl.ds(0, sz)],
                        out.at[pl.ds(
                            pl.multiple_of(bu * _BS + wrow + o, 8), sz)])

    return k


def _bucket_scatter(contrib, dsti, n):
    e = contrib.shape[0]
    nbuk = (n + _BS - 1) // _BS
    rout, bout, cout = _partition_call(e, nbuk)(contrib, dsti)
    return _bucket_acc_call(nbuk)(rout, bout, cout)


def _egc_fused(p, srci, dsti, pieces_n, pieces_e, stats_of, n_nodes, e_edges,
               bucketed, write_m):
    stn = [stats_of(a) for a in pieces_n]
    W1 = jnp.concatenate([p['W_src_gate'], p['W_dst_update']], axis=1)
    W2 = jnp.concatenate([p['W_dst_gate'],
                          jnp.zeros_like(p['W_dst_gate'])], axis=1)
    tbl_src, tbl_dst, xwsu = _mm(pieces_n, stn, p['bn_n_g'], p['bn_n_b'],
                                 [W1, W2, p['W_src_update']])
    ste = [stats_of(a) for a in pieces_e]
    (ye,) = _mm(pieces_e, ste, p['bn_e_g'], p['bn_e_b'], [p['W_edge_gate']])
    ec = _edge_compute_call(e_edges, write_m)
    if write_m:
        m, contrib = ec(tbl_src, tbl_dst, ye, srci, dsti)
    else:
        out = ec(tbl_src, tbl_dst, ye, srci, dsti)
        contrib = out[0] if isinstance(out, (list, tuple)) else out
        m = None
    if bucketed:
        acc = _edge_scatter_call(e_edges, 13440, 6)(contrib, dsti)
    else:
        acc = _edge_scatter_call(e_edges, 5120, 1)(contrib, dsti)
    BR = 2000 if n_nodes % 2000 == 0 else 1000
    x_out = _combine_call(n_nodes, BR, acc.shape[0])(xwsu, acc)
    return x_out, m


def kernel(x, y, z, edge_index, lg_edge_index, params):
    src = edge_index[0].astype(jnp.int32)
    dst = edge_index[1].astype(jnp.int32)
    lsrc = lg_edge_index[0].astype(jnp.int32)
    ldst = lg_edge_index[1].astype(jnp.int32)
    N = x.shape[0]
    E = y.shape[0]
    T = z.shape[0]
    xs, ys, zs = [x], [y], [z]
    stats_cache = {}

    def stats_of(a):
        key = id(a)
        if key not in stats_cache:
            stats_cache[key] = _stats(a)
        return stats_cache[key]

    for i in range(_L):
        lp = params['layers'][i]
        nx, ny = _egc_fused(lp['node_update'], src, dst, xs, ys, stats_of,
                            N, E, bucketed=False, write_m=True)
        ny2, nz = _egc_fused(lp['edge_update'], lsrc, ldst, [ny], zs,
                             stats_of, E, T, bucketed=True,
                             write_m=(i < _L - 1))
        xs.append(nx)
        ys.append(ny2)
        if nz is not None:
            zs.append(nz)

    bx = params['bottleneck_x']
    by = params['bottleneck_y']
    (x_out,) = _mm(xs, [stats_of(a) for a in xs], bx['g'], bx['b'],
                   [bx['W']], residual=x)
    (y_out,) = _mm(ys, [stats_of(a) for a in ys], by['g'], by['b'],
                   [by['W']], residual=y)
    return (x_out, y_out)


# speedup vs baseline: 2.1342x; 1.0719x over previous
import functools

import jax
import jax.numpy as jnp
from jax import lax
from jax.experimental import pallas as pl
from jax.experimental.pallas import tpu as pltpu
from jax.experimental.pallas import tpu_sc as plsc

_EPS_BN = 1e-5
_EPS_SEG = 1e-6
_C = 128
_NW = 32
_L = 3


@functools.lru_cache(None)
def _stats_call(R, F, BR):
    def body(a_ref, o_ref):
        a = a_ref[...]

        @pl.when(pl.program_id(0) == 0)
        def _():
            o_ref[...] = jnp.zeros_like(o_ref)

        o_ref[0:1, :] += jnp.sum(a, axis=0, keepdims=True)
        o_ref[1:2, :] += jnp.sum(a * a, axis=0, keepdims=True)

    return pl.pallas_call(
        body,
        grid=(R // BR,),
        in_specs=[pl.BlockSpec((BR, F), lambda i: (i, 0))],
        out_specs=pl.BlockSpec((8, F), lambda i: (0, 0)),
        out_shape=jax.ShapeDtypeStruct((8, F), jnp.float32),
    )


def _stats(a):
    R, F = a.shape
    BR = 2000 if R % 2000 == 0 else 1000
    return _stats_call(R, F, BR)(a)


@functools.lru_cache(None)
def _mm_call(R, BR, npieces, Ks, residual):
    nouts = len(Ks)

    def body(*refs):
        it = iter(refs)
        a = [next(it) for _ in range(npieces)]
        st = [next(it) for _ in range(npieces)]
        g = [next(it) for _ in range(npieces)]
        b = [next(it) for _ in range(npieces)]
        W = [[next(it) for _ in range(npieces)] for _ in range(nouts)]
        res = next(it) if residual else None
        outs = [next(it) for _ in range(nouts)]
        acc = [None] * nouts
        inv_r = 1.0 / R
        for j in range(npieces):
            aj = a[j][...]
            mean = st[j][0:1, :] * inv_r
            var = st[j][1:2, :] * inv_r - mean * mean
            xn = (aj - mean) * lax.rsqrt(var + _EPS_BN) * g[j][...] + b[j][...]
            xn = xn * jax.nn.sigmoid(xn)
            for o in range(nouts):
                d = jnp.dot(xn, W[o][j][...], preferred_element_type=jnp.float32)
                acc[o] = d if acc[o] is None else acc[o] + d
        for o in range(nouts):
            val = acc[o]
            if residual and o == 0:
                val = val + res[...]
            outs[o][...] = val

    in_specs = (
        [pl.BlockSpec((BR, 64), lambda i: (i, 0)) for _ in range(npieces)]
        + [pl.BlockSpec((8, 64), lambda i: (0, 0)) for _ in range(npieces)]
        + [pl.BlockSpec((1, 64), lambda i: (0, 0)) for _ in range(2 * npieces)]
        + [pl.BlockSpec((64, K), lambda i: (0, 0))
           for K in Ks for _ in range(npieces)]
        + ([pl.BlockSpec((BR, Ks[0]), lambda i: (i, 0))] if residual else [])
    )
    return pl.pallas_call(
        body,
        grid=(R // BR,),
        in_specs=in_specs,
        out_specs=[pl.BlockSpec((BR, K), lambda i: (i, 0)) for K in Ks],
        out_shape=[jax.ShapeDtypeStruct((R, K), jnp.float32) for K in Ks],
    )


def _mm(pieces, stats, g, b, Ws, residual=None):
    R = pieces[0].shape[0]
    npieces = len(pieces)
    Ks = tuple(int(W.shape[1]) for W in Ws)
    BR = 2000 if R % 2000 == 0 else 1000
    args = list(pieces) + list(stats)
    for j in range(npieces):
        args.append(g[64 * j:64 * (j + 1)].reshape(1, 64))
    for j in range(npieces):
        args.append(b[64 * j:64 * (j + 1)].reshape(1, 64))
    for W in Ws:
        for j in range(npieces):
            args.append(W[64 * j:64 * (j + 1), :])
    if residual is not None:
        args.append(residual)
    out = _mm_call(R, BR, npieces, Ks, residual is not None)(*args)
    return out


@functools.lru_cache(None)
def _combine_call(R, BR, Racc):
    def body(su_ref, acc_ref, o_ref):
        accv = acc_ref[...]
        o_ref[...] = su_ref[...] + accv[:, 0:64] / (accv[:, 64:128] + _EPS_SEG)

    return pl.pallas_call(
        body,
        grid=(R // BR,),
        in_specs=[pl.BlockSpec((BR, 64), lambda i: (i, 0)),
                  pl.BlockSpec((BR, 128), lambda i: (i, 0))],
        out_specs=pl.BlockSpec((BR, 64), lambda i: (i, 0)),
        out_shape=jax.ShapeDtypeStruct((R, 64), jnp.float32),
    )


@functools.lru_cache(None)
def _edge_compute_call(e, write_m):
    nchunk = e // _C
    mesh = plsc.VectorSubcoreMesh(core_axis_name="c", subcore_axis_name="s")
    out_type = [jax.ShapeDtypeStruct((e, 128), jnp.float32)]
    if write_m:
        out_type = [jax.ShapeDtypeStruct((e, 64), jnp.float32)] + out_type
    scratch = [
        pltpu.VMEM((_C,), jnp.int32), pltpu.VMEM((_C,), jnp.int32),
        pltpu.VMEM((_C,), jnp.int32), pltpu.VMEM((_C,), jnp.int32),
        pltpu.VMEM((_C, 128), jnp.float32), pltpu.VMEM((_C, 128), jnp.float32),
        pltpu.VMEM((_C, 128), jnp.float32), pltpu.VMEM((_C, 128), jnp.float32),
        pltpu.VMEM((_C, 64), jnp.float32), pltpu.VMEM((_C, 64), jnp.float32),
        pltpu.VMEM((_C, 128), jnp.float32),
        pltpu.SemaphoreType.DMA, pltpu.SemaphoreType.DMA,
    ]

    @functools.partial(pl.kernel, mesh=mesh, out_type=out_type,
                       scratch_types=scratch)
    def k(tbl_src, tbl_dst, ef, src, dst, *rest):
        ri = iter(rest)
        m_out = next(ri) if write_m else None
        c_out = next(ri)
        idxs_v = [next(ri), next(ri)]
        idxd_v = [next(ri), next(ri)]
        gsrc_v = [next(ri), next(ri)]
        gdst_v = [next(ri), next(ri)]
        ef_v = next(ri)
        m_v = next(ri)
        co_v = next(ri)
        gsem = [next(ri), next(ri)]
        wid = lax.axis_index("s") * 2 + lax.axis_index("c")
        c0 = wid * nchunk // _NW
        c1 = (wid + 1) * nchunk // _NW

        def start(ci, b):
            base = ci * _C
            pltpu.sync_copy(src.at[pl.ds(base, _C)], idxs_v[b])
            pltpu.sync_copy(dst.at[pl.ds(base, _C)], idxd_v[b])
            pltpu.async_copy(tbl_src.at[idxs_v[b]], gsrc_v[b], gsem[b])
            pltpu.async_copy(tbl_dst.at[idxd_v[b]], gdst_v[b], gsem[b])

        start(c0, 0)

        def chunk(ci, carry):
            par = lax.rem(ci - c0, 2)
            for b in (0, 1):
                @pl.when(par == b)
                def _():
                    @pl.when(ci + 1 < c1)
                    def _():
                        start(ci + 1, 1 - b)
                    pltpu.make_async_copy(
                        tbl_src.at[idxs_v[b]], gsrc_v[b], gsem[b]).wait()
                    pltpu.make_async_copy(
                        tbl_dst.at[idxd_v[b]], gdst_v[b], gsem[b]).wait()
                    pltpu.sync_copy(ef.at[pl.ds(ci * _C, _C)], ef_v)

                    def row(r, c2):
                        for k4 in range(4):
                            sl = pl.ds(k4 * 16, 16)
                            sh = pl.ds(64 + k4 * 16, 16)
                            mv = (gsrc_v[b][r, sl] + gdst_v[b][r, sl]
                                  + ef_v[r, sl])
                            sg = 1.0 / (1.0 + jnp.exp(-mv))
                            m_v[r, sl] = mv
                            co_v[r, sl] = gsrc_v[b][r, sh] * sg
                            co_v[r, sh] = sg
                        return c2

                    lax.fori_loop(0, _C, row, 0)
                    base = ci * _C
                    if write_m:
                        pltpu.sync_copy(m_v, m_out.at[pl.ds(base, _C)])
                    pltpu.sync_copy(co_v, c_out.at[pl.ds(base, _C)])
            return carry

        lax.fori_loop(c0, c1, chunk, 0)

    return k


_CS = 32


@functools.lru_cache(None)
def _edge_scatter_call(e, S, P):
    nchunk = e // _CS
    npad = 2 * P * S
    rpt = S // 16
    pieces = []
    off = 0
    while off < rpt:
        sz = min(16, rpt - off)
        pieces.append((off, sz))
        off += sz
    mesh = plsc.VectorSubcoreMesh(core_axis_name="c", subcore_axis_name="s")
    scratch = [
        pltpu.VMEM((_CS,), jnp.int32), pltpu.VMEM((_CS,), jnp.int32),
        pltpu.VMEM((_CS,), jnp.int32), pltpu.VMEM((_CS,), jnp.int32),
        pltpu.VMEM((_CS, 128), jnp.float32),
        pltpu.VMEM((_CS, 128), jnp.float32),
        pltpu.VMEM((16, 128), jnp.float32),
        pltpu.VMEM((16, 128), jnp.float32),
        pltpu.VMEM_SHARED((S + 16, 128), jnp.float32),
        pltpu.SemaphoreType.DMA, pltpu.SemaphoreType.DMA,
        pltpu.SemaphoreType.DMA, pltpu.SemaphoreType.DMA,
    ]

    @functools.partial(pl.kernel, mesh=mesh,
                       out_type=jax.ShapeDtypeStruct((npad, 128), jnp.float32),
                       scratch_types=scratch)
    def k(contrib, dstidx, out, i0, i1, l0, l1, cv0, cv1, z_v, w_v, acc_sp,
          sm0, sm1, am0, am1):
        idxd_v = [i0, i1]
        idxl_v = [l0, l1]
        co_v = [cv0, cv1]
        lsem = [sm0, sm1]
        asem = [am0, am1]
        core = lax.axis_index("c")
        sid = lax.axis_index("s")

        def start(ci, b):
            base = ci * _CS
            pltpu.async_copy(dstidx.at[pl.ds(base, _CS)], idxd_v[b], lsem[b])
            pltpu.async_copy(contrib.at[pl.ds(base, _CS)], co_v[b], lsem[b])

        def zrow(r, c2):
            for k4 in range(8):
                z_v[r, pl.ds(k4 * 16, 16)] = jnp.zeros((16,), jnp.float32)
            return c2

        lax.fori_loop(0, 16, zrow, 0)
        c0 = sid * nchunk // 16
        c1 = (sid + 1) * nchunk // 16
        for p in range(P):
            base_seg = pl.multiple_of((2 * p + core) * S, 8)
            row0 = pl.multiple_of(sid * rpt, 8)
            for (o, sz) in pieces:
                pltpu.sync_copy(z_v.at[pl.ds(0, sz)],
                                acc_sp.at[pl.ds(row0 + o, sz)])

            @pl.when(sid == 0)
            def _():
                pltpu.sync_copy(z_v.at[pl.ds(0, 16)], acc_sp.at[pl.ds(S, 16)])

            plsc.subcore_barrier()
            start(c0, 0)

            def chunk(ci, carry):
                par = lax.rem(ci - c0, 2)
                for b in (0, 1):
                    @pl.when(par == b)
                    def _():
                        @pl.when(ci > c0)
                        def _():
                            pltpu.make_async_copy(
                                co_v[1 - b], acc_sp.at[idxl_v[1 - b]],
                                asem[1 - b]).wait()

                        @pl.when(ci + 1 < c1)
                        def _():
                            start(ci + 1, 1 - b)
                        pltpu.make_async_copy(
                            dstidx.at[pl.ds(0, _CS)], idxd_v[b],
                            lsem[b]).wait()
                        pltpu.make_async_copy(
                            contrib.at[pl.ds(0, _CS)], co_v[b],
                            lsem[b]).wait()
                        for k8 in range(_CS // 16):
                            sl = pl.ds(k8 * 16, 16)
                            iv = idxd_v[b][sl] - base_seg
                            ok = (iv >= 0) & (iv < S)
                            idxl_v[b][sl] = jnp.where(ok, iv, S + sid)
                        pltpu.async_copy(co_v[b], acc_sp.at[idxl_v[b]],
                                        asem[b], add=True)
                return carry

            lax.fori_loop(c0, c1, chunk, 0)
            last_par = lax.rem(c1 - 1 - c0, 2)
            for b in (0, 1):
                @pl.when(last_par == b)
                def _():
                    pltpu.make_async_copy(co_v[b], acc_sp.at[idxl_v[b]],
                                          asem[b]).wait()
            plsc.subcore_barrier()
            for (o, sz) in pieces:
                pltpu.sync_copy(acc_sp.at[pl.ds(row0 + o, sz)],
                                w_v.at[pl.ds(0, sz)])
                pltpu.sync_copy(w_v.at[pl.ds(0, sz)],
                                out.at[pl.ds(pl.multiple_of(base_seg + row0 + o, 8), sz)])

    return k


_BS = 8192


def _take16(v, idx):
    dnums = lax.GatherDimensionNumbers(
        offset_dims=(), collapsed_slice_dims=(0,), start_index_map=(0,))
    return lax.gather(v, idx[:, None], dnums, slice_sizes=(1,),
                      mode=lax.GatherScatterMode.PROMISE_IN_BOUNDS)
_CAP = 1024


@functools.lru_cache(None)
def _partition_call(e, nbuk):
    nchunk = e // _C
    rows = _NW * nbuk * _CAP + 8
    trash = rows - 8
    big = 1 << 28
    mesh = plsc.VectorSubcoreMesh(core_axis_name="c", subcore_axis_name="s")
    out_type = [
        jax.ShapeDtypeStruct((rows, 128), jnp.float32),
        jax.ShapeDtypeStruct((_NW, nbuk * _CAP), jnp.int32),
        jax.ShapeDtypeStruct((_NW, 128), jnp.int32),
    ]
    scratch = [
        pltpu.VMEM((_C,), jnp.int32), pltpu.VMEM((_C,), jnp.int32),
        pltpu.VMEM((_C, 128), jnp.float32), pltpu.VMEM((_C, 128), jnp.float32),
        pltpu.VMEM((_C,), jnp.int32),
        pltpu.VMEM((nbuk * _CAP,), jnp.int32),
        pltpu.VMEM((128,), jnp.int32),
        pltpu.VMEM((128,), jnp.int32),
        pltpu.SemaphoreType.DMA, pltpu.SemaphoreType.DMA,
    ]

    @functools.partial(pl.kernel, mesh=mesh, out_type=out_type,
                       scratch_types=scratch)
    def k(contrib, dstidx, rout, bout, cout, i0, i1, cv0, cv1, gidx_v,
          stage_v, cflush_v, cnt_v, sm0, sm1):
        idxd_v = [i0, i1]
        co_v = [cv0, cv1]
        lsem = [sm0, sm1]
        wid = lax.axis_index("s") * 2 + lax.axis_index("c")
        c0 = wid * nchunk // _NW
        c1 = (wid + 1) * nchunk // _NW

        ones16 = jnp.ones((16,), jnp.int32)
        zeros16 = jnp.zeros((16,), jnp.int32)
        lanes = lax.iota(jnp.int32, 16)
        for j in range(8):
            cnt_v[pl.ds(j * 16, 16)] = zeros16
        for j in range(8):
            cflush_v[pl.ds(j * 16, 16)] = zeros16

        def initstage(j, carry):
            stage_v[pl.ds(j * 16, 16)] = zeros16 + big
            return carry

        lax.fori_loop(0, nbuk * _CAP // 16, initstage, 0)

        def start(ci, b):
            base = ci * _C
            pltpu.async_copy(dstidx.at[pl.ds(base, _C)], idxd_v[b], lsem[b])
            pltpu.async_copy(contrib.at[pl.ds(base, _C)], co_v[b], lsem[b])

        start(c0, 0)

        def chunk(ci, carry):
            par = lax.rem(ci - c0, 2)
            for b in (0, 1):
                @pl.when(par == b)
                def _():
                    @pl.when(ci + 1 < c1)
                    def _():
                        start(ci + 1, 1 - b)
                    pltpu.make_async_copy(
                        dstidx.at[pl.ds(0, _C)], idxd_v[b], lsem[b]).wait()
                    pltpu.make_async_copy(
                        contrib.at[pl.ds(0, _C)], co_v[b], lsem[b]).wait()
                    for k8 in range(8):
                        sl = pl.ds(k8 * 16, 16)
                        iv = idxd_v[b][sl]
                        bk = lax.shift_right_logical(iv, 13)
                        rank = zeros16
                        after = zeros16
                        for d in range(1, 16):
                            dn = _take16(bk, jnp.maximum(lanes - d, 0))
                            up = _take16(bk, jnp.minimum(lanes + d, 15))
                            rank = rank + jnp.where(
                                (dn == bk) & (lanes >= d), ones16, zeros16)
                            after = after + jnp.where(
                                (up == bk) & (lanes < 16 - d), ones16,
                                zeros16)
                        base_c = plsc.load_gather(cnt_v, [bk])
                        slot = base_c + rank
                        okc = slot < _CAP
                        plsc.store_scatter(cnt_v, [bk], slot + 1,
                                           mask=(after == 0))
                        g = wid * nbuk * _CAP + bk * _CAP + slot
                        gidx_v[sl] = jnp.where(okc, g, zeros16 + trash)
                        plsc.store_scatter(stage_v, [bk * _CAP + slot], iv,
                                           mask=okc)
                    pltpu.sync_copy(co_v[b], rout.at[gidx_v])
                return carry

        lax.fori_loop(c0, c1, chunk, 0)
        cflush_v[pl.ds(0, 16)] = cnt_v[pl.ds(0, 16)]
        cflush_v[pl.ds(16, 16)] = cnt_v[pl.ds(16, 16)]
        pltpu.sync_copy(stage_v, bout.at[wid])
        pltpu.sync_copy(cflush_v, cout.at[wid])

    return k


@functools.lru_cache(None)
def _bucket_acc_call(nbuk):
    sacc = _BS + 128
    rpt_z = sacc // 16
    rpt_w = _BS // 16
    mesh = plsc.VectorSubcoreMesh(core_axis_name="c", subcore_axis_name="s")
    scratch = [
        pltpu.VMEM((_C,), jnp.int32), pltpu.VMEM((_C,), jnp.int32),
        pltpu.VMEM((_C,), jnp.int32), pltpu.VMEM((_C,), jnp.int32),
        pltpu.VMEM((_C, 128), jnp.float32), pltpu.VMEM((_C, 128), jnp.float32),
        pltpu.VMEM((64, 128), jnp.float32),
        pltpu.VMEM((64, 128), jnp.float32),
        pltpu.VMEM((_NW, 128), jnp.int32),
        pltpu.VMEM_SHARED((sacc, 128), jnp.float32),
        pltpu.SemaphoreType.DMA, pltpu.SemaphoreType.DMA,
    ]

    @functools.partial(
        pl.kernel, mesh=mesh,
        out_type=jax.ShapeDtypeStruct((nbuk * _BS, 128), jnp.float32),
        scratch_types=scratch)
    def k(rowsbuf, bidx, counts, out, i0, i1, l0, l1, cv0, cv1, z_v, w_v,
          cnt_v, acc_sp, sm0, sm1):
        idxd_v = [i0, i1]
        idxl_v = [l0, l1]
        co_v = [cv0, cv1]
        lsem = [sm0, sm1]
        core = lax.axis_index("c")
        sid = lax.axis_index("s")
        pltpu.sync_copy(counts, cnt_v)

        def zrow(r, c2):
            for k4 in range(8):
                z_v[r, pl.ds(k4 * 16, 16)] = jnp.zeros((16,), jnp.float32)
            return c2

        lax.fori_loop(0, 64, zrow, 0)

        def start(regbase, j, b):
            base = regbase + j * _C
            pltpu.async_copy(rowsbuf.at[pl.ds(base, _C)], co_v[b], lsem[b])

        def startidx(pt, bu, j, b):
            pltpu.async_copy(
                bidx.at[pt, pl.ds(bu * _CAP + j * _C, _C)], idxd_v[b],
                lsem[b])

        for bu in range(nbuk):
            @pl.when(core == (bu % 2))
            def _():
                row0 = pl.multiple_of(sid * rpt_z, 8)
                for o in range(0, rpt_z, 64):
                    sz = min(64, rpt_z - o)
                    pltpu.sync_copy(z_v.at[pl.ds(0, sz)],
                                    acc_sp.at[pl.ds(row0 + o, sz)])
                plsc.subcore_barrier()
                for pt_off in (0, 16):
                    pt = sid + pt_off
                    cvec = cnt_v[pt, pl.ds((bu // 16) * 16, 16)]
                    cnt = cvec[bu % 16]
                    nch = jnp.minimum(
                        lax.shift_right_logical(cnt + (_C - 1), 7),
                        _CAP // _C)
                    regbase = (pt * nbuk + bu) * _CAP

                    @pl.when(nch > 0)
                    def _():
                        startidx(pt, bu, 0, 0)
                        start(regbase, 0, 0)

                        def chunk(j, carry):
                            par = lax.rem(j, 2)
                            for b in (0, 1):
                                @pl.when(par == b)
                                def _():
                                    @pl.when(j + 1 < nch)
                                    def _():
                                        startidx(pt, bu, j + 1, 1 - b)
                                        start(regbase, j + 1, 1 - b)
                                    pltpu.make_async_copy(
                                        bidx.at[0, pl.ds(0, _C)],
                                        idxd_v[b], lsem[b]).wait()
                                    pltpu.make_async_copy(
                                        rowsbuf.at[pl.ds(0, _C)],
                                        co_v[b], lsem[b]).wait()
                                    for k8 in range(8):
                                        sl = pl.ds(k8 * 16, 16)
                                        iv = idxd_v[b][sl] - bu * _BS
                                        ok = (iv >= 0) & (iv < _BS)
                                        idxl_v[b][sl] = jnp.where(
                                            ok, iv, _BS)
                                    pltpu.sync_copy(
                                        co_v[b], acc_sp.at[idxl_v[b]],
                                        add=True)
                            return carry

                        lax.fori_loop(0, nch, chunk, 0)
                plsc.subcore_barrier()
                wrow = pl.multiple_of(sid * rpt_w, 8)
                for o in range(0, rpt_w, 64):
                    sz = min(64, rpt_w - o)
                    pltpu.sync_copy(acc_sp.at[pl.ds(wrow + o, sz)],
                                    w_v.at[pl.ds(0, sz)])
                    pltpu.sync_copy(
                        w_v.at[pl.ds(0, sz)],
                        out.at[pl.ds(
                            pl.multiple_of(bu * _BS + wrow + o, 8), sz)])

    return k


def _bucket_scatter(contrib, dsti, n):
    e = contrib.shape[0]
    nbuk = (n + _BS - 1) // _BS
    rout, bout, cout = _partition_call(e, nbuk)(contrib, dsti)
    return _bucket_acc_call(nbuk)(rout, bout, cout)


def _egc_fused(p, srci, dsti, pieces_n, pieces_e, stats_of, n_nodes, e_edges,
               bucketed, write_m):
    stn = [stats_of(a) for a in pieces_n]
    W1 = jnp.concatenate([p['W_src_gate'], p['W_dst_update']], axis=1)
    W2 = jnp.concatenate([p['W_dst_gate'],
                          jnp.zeros_like(p['W_dst_gate'])], axis=1)
    tbl_src, tbl_dst, xwsu = _mm(pieces_n, stn, p['bn_n_g'], p['bn_n_b'],
                                 [W1, W2, p['W_src_update']])
    ste = [stats_of(a) for a in pieces_e]
    (ye,) = _mm(pieces_e, ste, p['bn_e_g'], p['bn_e_b'], [p['W_edge_gate']])
    ec = _edge_compute_call(e_edges, write_m)
    if write_m:
        m, contrib = ec(tbl_src, tbl_dst, ye, srci, dsti)
    else:
        out = ec(tbl_src, tbl_dst, ye, srci, dsti)
        contrib = out[0] if isinstance(out, (list, tuple)) else out
        m = None
    if bucketed:
        acc = _edge_scatter_call(e_edges, 13440, 6)(contrib, dsti)
    else:
        acc = _edge_scatter_call(e_edges, 5120, 1)(contrib, dsti)
    BR = 2000 if n_nodes % 2000 == 0 else 1000
    x_out = _combine_call(n_nodes, BR, acc.shape[0])(xwsu, acc)
    return x_out, m


def kernel(x, y, z, edge_index, lg_edge_index, params):
    src = edge_index[0].astype(jnp.int32)
    dst = edge_index[1].astype(jnp.int32)
    lsrc = lg_edge_index[0].astype(jnp.int32)
    ldst = lg_edge_index[1].astype(jnp.int32)
    N = x.shape[0]
    E = y.shape[0]
    T = z.shape[0]
    xs, ys, zs = [x], [y], [z]
    stats_cache = {}

    def stats_of(a):
        key = id(a)
        if key not in stats_cache:
            stats_cache[key] = _stats(a)
        return stats_cache[key]

    for i in range(_L):
        lp = params['layers'][i]
        nx, ny = _egc_fused(lp['node_update'], src, dst, xs, ys, stats_of,
                            N, E, bucketed=False, write_m=True)
        ny2, nz = _egc_fused(lp['edge_update'], lsrc, ldst, [ny], zs,
                             stats_of, E, T, bucketed=True,
                             write_m=(i < _L - 1))
        xs.append(nx)
        ys.append(ny2)
        if nz is not None:
            zs.append(nz)

    bx = params['bottleneck_x']
    by = params['bottleneck_y']
    (x_out,) = _mm(xs, [stats_of(a) for a in xs], bx['g'], bx['b'],
                   [bx['W']], residual=x)
    (y_out,) = _mm(ys, [stats_of(a) for a in ys], by['g'], by['b'],
                   [by['W']], residual=y)
    return (x_out, y_out)


# speedup vs baseline: 2.1701x; 1.0168x over previous
import functools

import jax
import jax.numpy as jnp
from jax import lax
from jax.experimental import pallas as pl
from jax.experimental.pallas import tpu as pltpu
from jax.experimental.pallas import tpu_sc as plsc

_EPS_BN = 1e-5
_EPS_SEG = 1e-6
_C = 128
_NW = 32
_L = 3


@functools.lru_cache(None)
def _stats_call(R, F, BR):
    def body(a_ref, o_ref):
        a = a_ref[...]

        @pl.when(pl.program_id(0) == 0)
        def _():
            o_ref[...] = jnp.zeros_like(o_ref)

        o_ref[0:1, :] += jnp.sum(a, axis=0, keepdims=True)
        o_ref[1:2, :] += jnp.sum(a * a, axis=0, keepdims=True)

    return pl.pallas_call(
        body,
        grid=(R // BR,),
        in_specs=[pl.BlockSpec((BR, F), lambda i: (i, 0))],
        out_specs=pl.BlockSpec((8, F), lambda i: (0, 0)),
        out_shape=jax.ShapeDtypeStruct((8, F), jnp.float32),
    )


def _stats(a):
    R, F = a.shape
    BR = 2000 if R % 2000 == 0 else 1000
    return _stats_call(R, F, BR)(a)


@functools.lru_cache(None)
def _mm_call(R, BR, npieces, Ks, residual):
    nouts = len(Ks)

    def body(*refs):
        it = iter(refs)
        a = [next(it) for _ in range(npieces)]
        st = [next(it) for _ in range(npieces)]
        g = [next(it) for _ in range(npieces)]
        b = [next(it) for _ in range(npieces)]
        W = [[next(it) for _ in range(npieces)] for _ in range(nouts)]
        res = next(it) if residual else None
        outs = [next(it) for _ in range(nouts)]
        acc = [None] * nouts
        inv_r = 1.0 / R
        for j in range(npieces):
            aj = a[j][...]
            mean = st[j][0:1, :] * inv_r
            var = st[j][1:2, :] * inv_r - mean * mean
            xn = (aj - mean) * lax.rsqrt(var + _EPS_BN) * g[j][...] + b[j][...]
            xn = xn * jax.nn.sigmoid(xn)
            for o in range(nouts):
                d = jnp.dot(xn, W[o][j][...], preferred_element_type=jnp.float32)
                acc[o] = d if acc[o] is None else acc[o] + d
        for o in range(nouts):
            val = acc[o]
            if residual and o == 0:
                val = val + res[...]
            outs[o][...] = val

    in_specs = (
        [pl.BlockSpec((BR, 64), lambda i: (i, 0)) for _ in range(npieces)]
        + [pl.BlockSpec((8, 64), lambda i: (0, 0)) for _ in range(npieces)]
        + [pl.BlockSpec((1, 64), lambda i: (0, 0)) for _ in range(2 * npieces)]
        + [pl.BlockSpec((64, K), lambda i: (0, 0))
           for K in Ks for _ in range(npieces)]
        + ([pl.BlockSpec((BR, Ks[0]), lambda i: (i, 0))] if residual else [])
    )
    return pl.pallas_call(
        body,
        grid=(R // BR,),
        in_specs=in_specs,
        out_specs=[pl.BlockSpec((BR, K), lambda i: (i, 0)) for K in Ks],
        out_shape=[jax.ShapeDtypeStruct((R, K), jnp.float32) for K in Ks],
    )


def _mm(pieces, stats, g, b, Ws, residual=None):
    R = pieces[0].shape[0]
    npieces = len(pieces)
    Ks = tuple(int(W.shape[1]) for W in Ws)
    BR = 2000 if R % 2000 == 0 else 1000
    args = list(pieces) + list(stats)
    for j in range(npieces):
        args.append(g[64 * j:64 * (j + 1)].reshape(1, 64))
    for j in range(npieces):
        args.append(b[64 * j:64 * (j + 1)].reshape(1, 64))
    for W in Ws:
        for j in range(npieces):
            args.append(W[64 * j:64 * (j + 1), :])
    if residual is not None:
        args.append(residual)
    out = _mm_call(R, BR, npieces, Ks, residual is not None)(*args)
    return out


@functools.lru_cache(None)
def _combine_call(R, BR, Racc):
    def body(su_ref, acc_ref, o_ref, st_ref):
        accv = acc_ref[...]
        nx = su_ref[...] + accv[:, 0:64] / (accv[:, 64:128] + _EPS_SEG)
        o_ref[...] = nx

        @pl.when(pl.program_id(0) == 0)
        def _():
            st_ref[...] = jnp.zeros_like(st_ref)

        st_ref[0:1, :] += jnp.sum(nx, axis=0, keepdims=True)
        st_ref[1:2, :] += jnp.sum(nx * nx, axis=0, keepdims=True)

    return pl.pallas_call(
        body,
        grid=(R // BR,),
        in_specs=[pl.BlockSpec((BR, 64), lambda i: (i, 0)),
                  pl.BlockSpec((BR, 128), lambda i: (i, 0))],
        out_specs=[pl.BlockSpec((BR, 64), lambda i: (i, 0)),
                   pl.BlockSpec((8, 64), lambda i: (0, 0))],
        out_shape=[jax.ShapeDtypeStruct((R, 64), jnp.float32),
                   jax.ShapeDtypeStruct((8, 64), jnp.float32)],
    )


@functools.lru_cache(None)
def _edge_compute_call(e, write_m):
    nchunk = e // _C
    mesh = plsc.VectorSubcoreMesh(core_axis_name="c", subcore_axis_name="s")
    out_type = [jax.ShapeDtypeStruct((e, 128), jnp.float32)]
    if write_m:
        out_type = [jax.ShapeDtypeStruct((e, 64), jnp.float32)] + out_type
    scratch = [
        pltpu.VMEM((_C,), jnp.int32), pltpu.VMEM((_C,), jnp.int32),
        pltpu.VMEM((_C,), jnp.int32), pltpu.VMEM((_C,), jnp.int32),
        pltpu.VMEM((_C, 128), jnp.float32), pltpu.VMEM((_C, 128), jnp.float32),
        pltpu.VMEM((_C, 128), jnp.float32), pltpu.VMEM((_C, 128), jnp.float32),
        pltpu.VMEM((_C, 64), jnp.float32), pltpu.VMEM((_C, 64), jnp.float32),
        pltpu.VMEM((_C, 128), jnp.float32),
        pltpu.SemaphoreType.DMA, pltpu.SemaphoreType.DMA,
    ]

    @functools.partial(pl.kernel, mesh=mesh, out_type=out_type,
                       scratch_types=scratch)
    def k(tbl_src, tbl_dst, ef, src, dst, *rest):
        ri = iter(rest)
        m_out = next(ri) if write_m else None
        c_out = next(ri)
        idxs_v = [next(ri), next(ri)]
        idxd_v = [next(ri), next(ri)]
        gsrc_v = [next(ri), next(ri)]
        gdst_v = [next(ri), next(ri)]
        ef_v = next(ri)
        m_v = next(ri)
        co_v = next(ri)
        gsem = [next(ri), next(ri)]
        wid = lax.axis_index("s") * 2 + lax.axis_index("c")
        c0 = wid * nchunk // _NW
        c1 = (wid + 1) * nchunk // _NW

        def start(ci, b):
            base = ci * _C
            pltpu.sync_copy(src.at[pl.ds(base, _C)], idxs_v[b])
            pltpu.sync_copy(dst.at[pl.ds(base, _C)], idxd_v[b])
            pltpu.async_copy(tbl_src.at[idxs_v[b]], gsrc_v[b], gsem[b])
            pltpu.async_copy(tbl_dst.at[idxd_v[b]], gdst_v[b], gsem[b])

        start(c0, 0)

        def chunk(ci, carry):
            par = lax.rem(ci - c0, 2)
            for b in (0, 1):
                @pl.when(par == b)
                def _():
                    @pl.when(ci + 1 < c1)
                    def _():
                        start(ci + 1, 1 - b)
                    pltpu.make_async_copy(
                        tbl_src.at[idxs_v[b]], gsrc_v[b], gsem[b]).wait()
                    pltpu.make_async_copy(
                        tbl_dst.at[idxd_v[b]], gdst_v[b], gsem[b]).wait()
                    pltpu.sync_copy(ef.at[pl.ds(ci * _C, _C)], ef_v)

                    def row(r, c2):
                        for k4 in range(4):
                            sl = pl.ds(k4 * 16, 16)
                            sh = pl.ds(64 + k4 * 16, 16)
                            mv = (gsrc_v[b][r, sl] + gdst_v[b][r, sl]
                                  + ef_v[r, sl])
                            sg = 1.0 / (1.0 + jnp.exp(-mv))
                            m_v[r, sl] = mv
                            co_v[r, sl] = gsrc_v[b][r, sh] * sg
                            co_v[r, sh] = sg
                        return c2

                    lax.fori_loop(0, _C, row, 0)
                    base = ci * _C
                    if write_m:
                        pltpu.sync_copy(m_v, m_out.at[pl.ds(base, _C)])
                    pltpu.sync_copy(co_v, c_out.at[pl.ds(base, _C)])
            return carry

        lax.fori_loop(c0, c1, chunk, 0)

    return k


_CS = 32


@functools.lru_cache(None)
def _edge_scatter_call(e, S, P):
    nchunk = e // _CS
    npad = 2 * P * S
    rpt = S // 16
    pieces = []
    off = 0
    while off < rpt:
        sz = min(16, rpt - off)
        pieces.append((off, sz))
        off += sz
    mesh = plsc.VectorSubcoreMesh(core_axis_name="c", subcore_axis_name="s")
    scratch = [
        pltpu.VMEM((_CS,), jnp.int32), pltpu.VMEM((_CS,), jnp.int32),
        pltpu.VMEM((_CS,), jnp.int32), pltpu.VMEM((_CS,), jnp.int32),
        pltpu.VMEM((_CS, 128), jnp.float32),
        pltpu.VMEM((_CS, 128), jnp.float32),
        pltpu.VMEM((16, 128), jnp.float32),
        pltpu.VMEM((16, 128), jnp.float32),
        pltpu.VMEM_SHARED((S + 16, 128), jnp.float32),
        pltpu.SemaphoreType.DMA, pltpu.SemaphoreType.DMA,
        pltpu.SemaphoreType.DMA, pltpu.SemaphoreType.DMA,
    ]

    @functools.partial(pl.kernel, mesh=mesh,
                       out_type=jax.ShapeDtypeStruct((npad, 128), jnp.float32),
                       scratch_types=scratch)
    def k(contrib, dstidx, out, i0, i1, l0, l1, cv0, cv1, z_v, w_v, acc_sp,
          sm0, sm1, am0, am1):
        idxd_v = [i0, i1]
        idxl_v = [l0, l1]
        co_v = [cv0, cv1]
        lsem = [sm0, sm1]
        asem = [am0, am1]
        core = lax.axis_index("c")
        sid = lax.axis_index("s")

        def start(ci, b):
            base = ci * _CS
            pltpu.async_copy(dstidx.at[pl.ds(base, _CS)], idxd_v[b], lsem[b])
            pltpu.async_copy(contrib.at[pl.ds(base, _CS)], co_v[b], lsem[b])

        def zrow(r, c2):
            for k4 in range(8):
                z_v[r, pl.ds(k4 * 16, 16)] = jnp.zeros((16,), jnp.float32)
            return c2

        lax.fori_loop(0, 16, zrow, 0)
        c0 = sid * nchunk // 16
        c1 = (sid + 1) * nchunk // 16
        for p in range(P):
            base_seg = pl.multiple_of((2 * p + core) * S, 8)
            row0 = pl.multiple_of(sid * rpt, 8)
            for (o, sz) in pieces:
                pltpu.sync_copy(z_v.at[pl.ds(0, sz)],
                                acc_sp.at[pl.ds(row0 + o, sz)])

            @pl.when(sid == 0)
            def _():
                pltpu.sync_copy(z_v.at[pl.ds(0, 16)], acc_sp.at[pl.ds(S, 16)])

            plsc.subcore_barrier()
            start(c0, 0)

            def chunk(ci, carry):
                par = lax.rem(ci - c0, 2)
                for b in (0, 1):
                    @pl.when(par == b)
                    def _():
                        @pl.when(ci > c0)
                        def _():
                            pltpu.make_async_copy(
                                co_v[1 - b], acc_sp.at[idxl_v[1 - b]],
                                asem[1 - b]).wait()

                        @pl.when(ci + 1 < c1)
                        def _():
                            start(ci + 1, 1 - b)
                        pltpu.make_async_copy(
                            dstidx.at[pl.ds(0, _CS)], idxd_v[b],
                            lsem[b]).wait()
                        pltpu.make_async_copy(
                            contrib.at[pl.ds(0, _CS)], co_v[b],
                            lsem[b]).wait()
                        for k8 in range(_CS // 16):
                            sl = pl.ds(k8 * 16, 16)
                            iv = idxd_v[b][sl] - base_seg
                            ok = (iv >= 0) & (iv < S)
                            idxl_v[b][sl] = jnp.where(ok, iv, S + sid)
                        pltpu.async_copy(co_v[b], acc_sp.at[idxl_v[b]],
                                        asem[b], add=True)
                return carry

            lax.fori_loop(c0, c1, chunk, 0)
            last_par = lax.rem(c1 - 1 - c0, 2)
            for b in (0, 1):
                @pl.when(last_par == b)
                def _():
                    pltpu.make_async_copy(co_v[b], acc_sp.at[idxl_v[b]],
                                          asem[b]).wait()
            plsc.subcore_barrier()
            for (o, sz) in pieces:
                pltpu.sync_copy(acc_sp.at[pl.ds(row0 + o, sz)],
                                w_v.at[pl.ds(0, sz)])
                pltpu.sync_copy(w_v.at[pl.ds(0, sz)],
                                out.at[pl.ds(pl.multiple_of(base_seg + row0 + o, 8), sz)])

    return k


_BS = 8192


def _take16(v, idx):
    dnums = lax.GatherDimensionNumbers(
        offset_dims=(), collapsed_slice_dims=(0,), start_index_map=(0,))
    return lax.gather(v, idx[:, None], dnums, slice_sizes=(1,),
                      mode=lax.GatherScatterMode.PROMISE_IN_BOUNDS)
_CAP = 1024


@functools.lru_cache(None)
def _partition_call(e, nbuk):
    nchunk = e // _C
    rows = _NW * nbuk * _CAP + 8
    trash = rows - 8
    big = 1 << 28
    mesh = plsc.VectorSubcoreMesh(core_axis_name="c", subcore_axis_name="s")
    out_type = [
        jax.ShapeDtypeStruct((rows, 128), jnp.float32),
        jax.ShapeDtypeStruct((_NW, nbuk * _CAP), jnp.int32),
        jax.ShapeDtypeStruct((_NW, 128), jnp.int32),
    ]
    scratch = [
        pltpu.VMEM((_C,), jnp.int32), pltpu.VMEM((_C,), jnp.int32),
        pltpu.VMEM((_C, 128), jnp.float32), pltpu.VMEM((_C, 128), jnp.float32),
        pltpu.VMEM((_C,), jnp.int32),
        pltpu.VMEM((nbuk * _CAP,), jnp.int32),
        pltpu.VMEM((128,), jnp.int32),
        pltpu.VMEM((128,), jnp.int32),
        pltpu.SemaphoreType.DMA, pltpu.SemaphoreType.DMA,
    ]

    @functools.partial(pl.kernel, mesh=mesh, out_type=out_type,
                       scratch_types=scratch)
    def k(contrib, dstidx, rout, bout, cout, i0, i1, cv0, cv1, gidx_v,
          stage_v, cflush_v, cnt_v, sm0, sm1):
        idxd_v = [i0, i1]
        co_v = [cv0, cv1]
        lsem = [sm0, sm1]
        wid = lax.axis_index("s") * 2 + lax.axis_index("c")
        c0 = wid * nchunk // _NW
        c1 = (wid + 1) * nchunk // _NW

        ones16 = jnp.ones((16,), jnp.int32)
        zeros16 = jnp.zeros((16,), jnp.int32)
        lanes = lax.iota(jnp.int32, 16)
        for j in range(8):
            cnt_v[pl.ds(j * 16, 16)] = zeros16
        for j in range(8):
            cflush_v[pl.ds(j * 16, 16)] = zeros16

        def initstage(j, carry):
            stage_v[pl.ds(j * 16, 16)] = zeros16 + big
            return carry

        lax.fori_loop(0, nbuk * _CAP // 16, initstage, 0)

        def start(ci, b):
            base = ci * _C
            pltpu.async_copy(dstidx.at[pl.ds(base, _C)], idxd_v[b], lsem[b])
            pltpu.async_copy(contrib.at[pl.ds(base, _C)], co_v[b], lsem[b])

        start(c0, 0)

        def chunk(ci, carry):
            par = lax.rem(ci - c0, 2)
            for b in (0, 1):
                @pl.when(par == b)
                def _():
                    @pl.when(ci + 1 < c1)
                    def _():
                        start(ci + 1, 1 - b)
                    pltpu.make_async_copy(
                        dstidx.at[pl.ds(0, _C)], idxd_v[b], lsem[b]).wait()
                    pltpu.make_async_copy(
                        contrib.at[pl.ds(0, _C)], co_v[b], lsem[b]).wait()
                    for k8 in range(8):
                        sl = pl.ds(k8 * 16, 16)
                        iv = idxd_v[b][sl]
                        bk = lax.shift_right_logical(iv, 13)
                        rank = zeros16
                        after = zeros16
                        for d in range(1, 16):
                            dn = _take16(bk, jnp.maximum(lanes - d, 0))
                            up = _take16(bk, jnp.minimum(lanes + d, 15))
                            rank = rank + jnp.where(
                                (dn == bk) & (lanes >= d), ones16, zeros16)
                            after = after + jnp.where(
                                (up == bk) & (lanes < 16 - d), ones16,
                                zeros16)
                        base_c = plsc.load_gather(cnt_v, [bk])
                        slot = base_c + rank
                        okc = slot < _CAP
                        plsc.store_scatter(cnt_v, [bk], slot + 1,
                                           mask=(after == 0))
                        g = wid * nbuk * _CAP + bk * _CAP + slot
                        gidx_v[sl] = jnp.where(okc, g, zeros16 + trash)
                        plsc.store_scatter(stage_v, [bk * _CAP + slot], iv,
                                           mask=okc)
                    pltpu.sync_copy(co_v[b], rout.at[gidx_v])
                return carry

        lax.fori_loop(c0, c1, chunk, 0)
        cflush_v[pl.ds(0, 16)] = cnt_v[pl.ds(0, 16)]
        cflush_v[pl.ds(16, 16)] = cnt_v[pl.ds(16, 16)]
        pltpu.sync_copy(stage_v, bout.at[wid])
        pltpu.sync_copy(cflush_v, cout.at[wid])

    return k


@functools.lru_cache(None)
def _bucket_acc_call(nbuk):
    sacc = _BS + 128
    rpt_z = sacc // 16
    rpt_w = _BS // 16
    mesh = plsc.VectorSubcoreMesh(core_axis_name="c", subcore_axis_name="s")
    scratch = [
        pltpu.VMEM((_C,), jnp.int32), pltpu.VMEM((_C,), jnp.int32),
        pltpu.VMEM((_C,), jnp.int32), pltpu.VMEM((_C,), jnp.int32),
        pltpu.VMEM((_C, 128), jnp.float32), pltpu.VMEM((_C, 128), jnp.float32),
        pltpu.VMEM((64, 128), jnp.float32),
        pltpu.VMEM((64, 128), jnp.float32),
        pltpu.VMEM((_NW, 128), jnp.int32),
        pltpu.VMEM_SHARED((sacc, 128), jnp.float32),
        pltpu.SemaphoreType.DMA, pltpu.SemaphoreType.DMA,
    ]

    @functools.partial(
        pl.kernel, mesh=mesh,
        out_type=jax.ShapeDtypeStruct((nbuk * _BS, 128), jnp.float32),
        scratch_types=scratch)
    def k(rowsbuf, bidx, counts, out, i0, i1, l0, l1, cv0, cv1, z_v, w_v,
          cnt_v, acc_sp, sm0, sm1):
        idxd_v = [i0, i1]
        idxl_v = [l0, l1]
        co_v = [cv0, cv1]
        lsem = [sm0, sm1]
        core = lax.axis_index("c")
        sid = lax.axis_index("s")
        pltpu.sync_copy(counts, cnt_v)

        def zrow(r, c2):
            for k4 in range(8):
                z_v[r, pl.ds(k4 * 16, 16)] = jnp.zeros((16,), jnp.float32)
            return c2

        lax.fori_loop(0, 64, zrow, 0)

        def start(regbase, j, b):
            base = regbase + j * _C
            pltpu.async_copy(rowsbuf.at[pl.ds(base, _C)], co_v[b], lsem[b])

        def startidx(pt, bu, j, b):
            pltpu.async_copy(
                bidx.at[pt, pl.ds(bu * _CAP + j * _C, _C)], idxd_v[b],
                lsem[b])

        for bu in range(nbuk):
            @pl.when(core == (bu % 2))
            def _():
                row0 = pl.multiple_of(sid * rpt_z, 8)
                for o in range(0, rpt_z, 64):
                    sz = min(64, rpt_z - o)
                    pltpu.sync_copy(z_v.at[pl.ds(0, sz)],
                                    acc_sp.at[pl.ds(row0 + o, sz)])
                plsc.subcore_barrier()
                for pt_off in (0, 16):
                    pt = sid + pt_off
                    cvec = cnt_v[pt, pl.ds((bu // 16) * 16, 16)]
                    cnt = cvec[bu % 16]
                    nch = jnp.minimum(
                        lax.shift_right_logical(cnt + (_C - 1), 7),
                        _CAP // _C)
                    regbase = (pt * nbuk + bu) * _CAP

                    @pl.when(nch > 0)
                    def _():
                        startidx(pt, bu, 0, 0)
                        start(regbase, 0, 0)

                        def chunk(j, carry):
                            par = lax.rem(j, 2)
                            for b in (0, 1):
                                @pl.when(par == b)
                                def _():
                                    @pl.when(j + 1 < nch)
                                    def _():
                                        startidx(pt, bu, j + 1, 1 - b)
                                        start(regbase, j + 1, 1 - b)
                                    pltpu.make_async_copy(
                                        bidx.at[0, pl.ds(0, _C)],
                                        idxd_v[b], lsem[b]).wait()
                                    pltpu.make_async_copy(
                                        rowsbuf.at[pl.ds(0, _C)],
                                        co_v[b], lsem[b]).wait()
                                    for k8 in range(8):
                                        sl = pl.ds(k8 * 16, 16)
                                        iv = idxd_v[b][sl] - bu * _BS
                                        ok = (iv >= 0) & (iv < _BS)
                                        idxl_v[b][sl] = jnp.where(
                                            ok, iv, _BS)
                                    pltpu.sync_copy(
                                        co_v[b], acc_sp.at[idxl_v[b]],
                                        add=True)
                            return carry

                        lax.fori_loop(0, nch, chunk, 0)
                plsc.subcore_barrier()
                wrow = pl.multiple_of(sid * rpt_w, 8)
                for o in range(0, rpt_w, 64):
                    sz = min(64, rpt_w - o)
                    pltpu.sync_copy(acc_sp.at[pl.ds(wrow + o, sz)],
                                    w_v.at[pl.ds(0, sz)])
                    pltpu.sync_copy(
                        w_v.at[pl.ds(0, sz)],
                        out.at[pl.ds(
                            pl.multiple_of(bu * _BS + wrow + o, 8), sz)])

    return k


def _bucket_scatter(contrib, dsti, n):
    e = contrib.shape[0]
    nbuk = (n + _BS - 1) // _BS
    rout, bout, cout = _partition_call(e, nbuk)(contrib, dsti)
    return _bucket_acc_call(nbuk)(rout, bout, cout)


def _egc_fused(p, srci, dsti, pieces_n, pieces_e, stats_of, n_nodes, e_edges,
               bucketed, write_m):
    stn = [stats_of(a) for a in pieces_n]
    W1 = jnp.concatenate([p['W_src_gate'], p['W_dst_update']], axis=1)
    W2 = jnp.concatenate([p['W_dst_gate'],
                          jnp.zeros_like(p['W_dst_gate'])], axis=1)
    tbl_src, tbl_dst, xwsu = _mm(pieces_n, stn, p['bn_n_g'], p['bn_n_b'],
                                 [W1, W2, p['W_src_update']])
    ste = [stats_of(a) for a in pieces_e]
    (ye,) = _mm(pieces_e, ste, p['bn_e_g'], p['bn_e_b'], [p['W_edge_gate']])
    ec = _edge_compute_call(e_edges, write_m)
    if write_m:
        m, contrib = ec(tbl_src, tbl_dst, ye, srci, dsti)
    else:
        out = ec(tbl_src, tbl_dst, ye, srci, dsti)
        contrib = out[0] if isinstance(out, (list, tuple)) else out
        m = None
    if bucketed:
        acc = _edge_scatter_call(e_edges, 13440, 6)(contrib, dsti)
    else:
        acc = _edge_scatter_call(e_edges, 5120, 1)(contrib, dsti)
    BR = 2000 if n_nodes % 2000 == 0 else 1000
    x_out, x_st = _combine_call(n_nodes, BR, acc.shape[0])(xwsu, acc)
    return x_out, x_st, m


def kernel(x, y, z, edge_index, lg_edge_index, params):
    src = edge_index[0].astype(jnp.int32)
    dst = edge_index[1].astype(jnp.int32)
    lsrc = lg_edge_index[0].astype(jnp.int32)
    ldst = lg_edge_index[1].astype(jnp.int32)
    N = x.shape[0]
    E = y.shape[0]
    T = z.shape[0]
    xs, ys, zs = [x], [y], [z]
    stats_cache = {}

    def stats_of(a):
        key = id(a)
        if key not in stats_cache:
            stats_cache[key] = _stats(a)
        return stats_cache[key]

    for i in range(_L):
        lp = params['layers'][i]
        nx, nx_st, ny = _egc_fused(lp['node_update'], src, dst, xs, ys,
                                   stats_of, N, E, bucketed=False,
                                   write_m=True)
        stats_cache[id(nx)] = nx_st
        ny2, ny2_st, nz = _egc_fused(lp['edge_update'], lsrc, ldst, [ny], zs,
                                     stats_of, E, T, bucketed=True,
                                     write_m=(i < _L - 1))
        stats_cache[id(ny2)] = ny2_st
        xs.append(nx)
        ys.append(ny2)
        if nz is not None:
            zs.append(nz)

    bx = params['bottleneck_x']
    by = params['bottleneck_y']
    (x_out,) = _mm(xs, [stats_of(a) for a in xs], bx['g'], bx['b'],
                   [bx['W']], residual=x)
    (y_out,) = _mm(ys, [stats_of(a) for a in ys], by['g'], by['b'],
                   [by['W']], residual=y)
    return (x_out, y_out)


# speedup vs baseline: 2.2782x; 1.0498x over previous
import functools

import jax
import jax.numpy as jnp
from jax import lax
from jax.experimental import pallas as pl
from jax.experimental.pallas import tpu as pltpu
from jax.experimental.pallas import tpu_sc as plsc

_EPS_BN = 1e-5
_EPS_SEG = 1e-6
_C = 128
_NW = 32
_L = 3


@functools.lru_cache(None)
def _stats_call(R, F, BR):
    def body(a_ref, o_ref):
        a = a_ref[...]

        @pl.when(pl.program_id(0) == 0)
        def _():
            o_ref[...] = jnp.zeros_like(o_ref)

        o_ref[0:1, :] += jnp.sum(a, axis=0, keepdims=True)
        o_ref[1:2, :] += jnp.sum(a * a, axis=0, keepdims=True)

    return pl.pallas_call(
        body,
        grid=(R // BR,),
        in_specs=[pl.BlockSpec((BR, F), lambda i: (i, 0))],
        out_specs=pl.BlockSpec((8, F), lambda i: (0, 0)),
        out_shape=jax.ShapeDtypeStruct((8, F), jnp.float32),
    )


def _stats(a):
    R, F = a.shape
    BR = 2000 if R % 2000 == 0 else 1000
    return _stats_call(R, F, BR)(a)


@functools.lru_cache(None)
def _mm_call(R, BR, npieces, Ks, residual):
    nouts = len(Ks)

    def body(*refs):
        it = iter(refs)
        a = [next(it) for _ in range(npieces)]
        st = [next(it) for _ in range(npieces)]
        g = [next(it) for _ in range(npieces)]
        b = [next(it) for _ in range(npieces)]
        W = [[next(it) for _ in range(npieces)] for _ in range(nouts)]
        res = next(it) if residual else None
        outs = [next(it) for _ in range(nouts)]
        acc = [None] * nouts
        inv_r = 1.0 / R
        for j in range(npieces):
            aj = a[j][...]
            mean = st[j][0:1, :] * inv_r
            var = st[j][1:2, :] * inv_r - mean * mean
            xn = (aj - mean) * lax.rsqrt(var + _EPS_BN) * g[j][...] + b[j][...]
            xn = xn * jax.nn.sigmoid(xn)
            for o in range(nouts):
                d = jnp.dot(xn, W[o][j][...], preferred_element_type=jnp.float32)
                acc[o] = d if acc[o] is None else acc[o] + d
        for o in range(nouts):
            val = acc[o]
            if residual and o == 0:
                val = val + res[...]
            outs[o][...] = val

    in_specs = (
        [pl.BlockSpec((BR, 64), lambda i: (i, 0)) for _ in range(npieces)]
        + [pl.BlockSpec((8, 64), lambda i: (0, 0)) for _ in range(npieces)]
        + [pl.BlockSpec((1, 64), lambda i: (0, 0)) for _ in range(2 * npieces)]
        + [pl.BlockSpec((64, K), lambda i: (0, 0))
           for K in Ks for _ in range(npieces)]
        + ([pl.BlockSpec((BR, Ks[0]), lambda i: (i, 0))] if residual else [])
    )
    return pl.pallas_call(
        body,
        grid=(R // BR,),
        in_specs=in_specs,
        out_specs=[pl.BlockSpec((BR, K), lambda i: (i, 0)) for K in Ks],
        out_shape=[jax.ShapeDtypeStruct((R, K), jnp.float32) for K in Ks],
    )


def _mm(pieces, stats, g, b, Ws, residual=None):
    R = pieces[0].shape[0]
    npieces = len(pieces)
    Ks = tuple(int(W.shape[1]) for W in Ws)
    BR = 2000 if R % 2000 == 0 else 1000
    args = list(pieces) + list(stats)
    for j in range(npieces):
        args.append(g[64 * j:64 * (j + 1)].reshape(1, 64))
    for j in range(npieces):
        args.append(b[64 * j:64 * (j + 1)].reshape(1, 64))
    for W in Ws:
        for j in range(npieces):
            args.append(W[64 * j:64 * (j + 1), :])
    if residual is not None:
        args.append(residual)
    out = _mm_call(R, BR, npieces, Ks, residual is not None)(*args)
    return out


@functools.lru_cache(None)
def _combine_call(R, BR, Racc):
    def body(su_ref, acc_ref, o_ref, st_ref):
        accv = acc_ref[...]
        nx = su_ref[...] + accv[:, 0:64] / (accv[:, 64:128] + _EPS_SEG)
        o_ref[...] = nx

        @pl.when(pl.program_id(0) == 0)
        def _():
            st_ref[...] = jnp.zeros_like(st_ref)

        st_ref[0:1, :] += jnp.sum(nx, axis=0, keepdims=True)
        st_ref[1:2, :] += jnp.sum(nx * nx, axis=0, keepdims=True)

    return pl.pallas_call(
        body,
        grid=(R // BR,),
        in_specs=[pl.BlockSpec((BR, 64), lambda i: (i, 0)),
                  pl.BlockSpec((BR, 128), lambda i: (i, 0))],
        out_specs=[pl.BlockSpec((BR, 64), lambda i: (i, 0)),
                   pl.BlockSpec((8, 64), lambda i: (0, 0))],
        out_shape=[jax.ShapeDtypeStruct((R, 64), jnp.float32),
                   jax.ShapeDtypeStruct((8, 64), jnp.float32)],
    )


@functools.lru_cache(None)
def _edge_compute_call(e, write_m):
    CE = 64
    nchunk = e // CE
    mesh = plsc.VectorSubcoreMesh(core_axis_name="c", subcore_axis_name="s")
    out_type = [jax.ShapeDtypeStruct((e, 128), jnp.float32)]
    if write_m:
        out_type = [jax.ShapeDtypeStruct((e, 64), jnp.float32)] + out_type
    scratch = [
        pltpu.VMEM((CE,), jnp.int32), pltpu.VMEM((CE,), jnp.int32),
        pltpu.VMEM((CE,), jnp.int32), pltpu.VMEM((CE,), jnp.int32),
        pltpu.VMEM((CE, 128), jnp.float32), pltpu.VMEM((CE, 128), jnp.float32),
        pltpu.VMEM((CE, 128), jnp.float32), pltpu.VMEM((CE, 128), jnp.float32),
        pltpu.VMEM((CE, 64), jnp.float32), pltpu.VMEM((CE, 64), jnp.float32),
        pltpu.VMEM((CE, 64), jnp.float32), pltpu.VMEM((CE, 64), jnp.float32),
        pltpu.VMEM((CE, 128), jnp.float32), pltpu.VMEM((CE, 128), jnp.float32),
        pltpu.SemaphoreType.DMA, pltpu.SemaphoreType.DMA,
        pltpu.SemaphoreType.DMA, pltpu.SemaphoreType.DMA,
        pltpu.SemaphoreType.DMA, pltpu.SemaphoreType.DMA,
    ]

    @functools.partial(pl.kernel, mesh=mesh, out_type=out_type,
                       scratch_types=scratch)
    def k(tbl_src, tbl_dst, ef, src, dst, *rest):
        ri = iter(rest)
        m_out = next(ri) if write_m else None
        c_out = next(ri)
        idxs_v = [next(ri), next(ri)]
        idxd_v = [next(ri), next(ri)]
        gsrc_v = [next(ri), next(ri)]
        gdst_v = [next(ri), next(ri)]
        ef_v = [next(ri), next(ri)]
        m_v = [next(ri), next(ri)]
        co_v = [next(ri), next(ri)]
        gsem = [next(ri), next(ri)]
        isem = [next(ri), next(ri)]
        wsem = [next(ri), next(ri)]
        wid = lax.axis_index("s") * 2 + lax.axis_index("c")
        c0 = wid * nchunk // _NW
        c1 = (wid + 1) * nchunk // _NW

        def start_idx(ci, b):
            base = ci * CE
            pltpu.async_copy(src.at[pl.ds(base, CE)], idxs_v[b], isem[b])
            pltpu.async_copy(dst.at[pl.ds(base, CE)], idxd_v[b], isem[b])

        def wait_idx(b):
            pltpu.make_async_copy(src.at[pl.ds(0, CE)], idxs_v[b],
                                  isem[b]).wait()
            pltpu.make_async_copy(dst.at[pl.ds(0, CE)], idxd_v[b],
                                  isem[b]).wait()

        def start_gather(ci, b):
            base = ci * CE
            pltpu.async_copy(tbl_src.at[idxs_v[b]], gsrc_v[b], gsem[b])
            pltpu.async_copy(tbl_dst.at[idxd_v[b]], gdst_v[b], gsem[b])
            pltpu.async_copy(ef.at[pl.ds(base, CE)], ef_v[b], gsem[b])

        def wait_gather(b):
            pltpu.make_async_copy(tbl_src.at[idxs_v[b]], gsrc_v[b],
                                  gsem[b]).wait()
            pltpu.make_async_copy(tbl_dst.at[idxd_v[b]], gdst_v[b],
                                  gsem[b]).wait()
            pltpu.make_async_copy(ef.at[pl.ds(0, CE)], ef_v[b],
                                  gsem[b]).wait()

        def wait_write(b):
            if write_m:
                pltpu.make_async_copy(m_v[b], m_out.at[pl.ds(0, CE)],
                                      wsem[b]).wait()
            pltpu.make_async_copy(co_v[b], c_out.at[pl.ds(0, CE)],
                                  wsem[b]).wait()

        start_idx(c0, 0)
        start_idx(c0 + 1, 1)
        wait_idx(0)
        start_gather(c0, 0)

        def chunk(ci, carry):
            par = lax.rem(ci - c0, 2)
            for b in (0, 1):
                @pl.when(par == b)
                def _():
                    @pl.when(ci + 1 < c1)
                    def _():
                        wait_idx(1 - b)
                        start_gather(ci + 1, 1 - b)
                    wait_gather(b)

                    @pl.when(ci + 2 < c1)
                    def _():
                        start_idx(ci + 2, b)

                    @pl.when(ci - 2 >= c0)
                    def _():
                        wait_write(b)

                    def row(r, c2):
                        for k4 in range(4):
                            sl = pl.ds(k4 * 16, 16)
                            sh = pl.ds(64 + k4 * 16, 16)
                            mv = (gsrc_v[b][r, sl] + gdst_v[b][r, sl]
                                  + ef_v[b][r, sl])
                            sg = 1.0 / (1.0 + jnp.exp(-mv))
                            m_v[b][r, sl] = mv
                            co_v[b][r, sl] = gsrc_v[b][r, sh] * sg
                            co_v[b][r, sh] = sg
                        return c2

                    lax.fori_loop(0, CE, row, 0)
                    base = ci * CE
                    if write_m:
                        pltpu.async_copy(m_v[b], m_out.at[pl.ds(base, CE)],
                                         wsem[b])
                    pltpu.async_copy(co_v[b], c_out.at[pl.ds(base, CE)],
                                     wsem[b])
            return carry

        lax.fori_loop(c0, c1, chunk, 0)
        last_par = lax.rem(c1 - 1 - c0, 2)
        for b in (0, 1):
            @pl.when(last_par == b)
            def _():
                wait_write(b)

            @pl.when((c1 - c0 >= 2) & (last_par == 1 - b))
            def _():
                wait_write(b)


    return k


_CS = 32


@functools.lru_cache(None)
def _edge_scatter_call(e, S, P):
    nchunk = e // _CS
    npad = 2 * P * S
    rpt = S // 16
    pieces = []
    off = 0
    while off < rpt:
        sz = min(16, rpt - off)
        pieces.append((off, sz))
        off += sz
    mesh = plsc.VectorSubcoreMesh(core_axis_name="c", subcore_axis_name="s")
    scratch = [
        pltpu.VMEM((_CS,), jnp.int32), pltpu.VMEM((_CS,), jnp.int32),
        pltpu.VMEM((_CS,), jnp.int32), pltpu.VMEM((_CS,), jnp.int32),
        pltpu.VMEM((_CS, 128), jnp.float32),
        pltpu.VMEM((_CS, 128), jnp.float32),
        pltpu.VMEM((16, 128), jnp.float32),
        pltpu.VMEM((16, 128), jnp.float32),
        pltpu.VMEM_SHARED((S + 16, 128), jnp.float32),
        pltpu.SemaphoreType.DMA, pltpu.SemaphoreType.DMA,
        pltpu.SemaphoreType.DMA, pltpu.SemaphoreType.DMA,
    ]

    @functools.partial(pl.kernel, mesh=mesh,
                       out_type=jax.ShapeDtypeStruct((npad, 128), jnp.float32),
                       scratch_types=scratch)
    def k(contrib, dstidx, out, i0, i1, l0, l1, cv0, cv1, z_v, w_v, acc_sp,
          sm0, sm1, am0, am1):
        idxd_v = [i0, i1]
        idxl_v = [l0, l1]
        co_v = [cv0, cv1]
        lsem = [sm0, sm1]
        asem = [am0, am1]
        core = lax.axis_index("c")
        sid = lax.axis_index("s")

        def start(ci, b):
            base = ci * _CS
            pltpu.async_copy(dstidx.at[pl.ds(base, _CS)], idxd_v[b], lsem[b])
            pltpu.async_copy(contrib.at[pl.ds(base, _CS)], co_v[b], lsem[b])

        def zrow(r, c2):
            for k4 in range(8):
                z_v[r, pl.ds(k4 * 16, 16)] = jnp.zeros((16,), jnp.float32)
            return c2

        lax.fori_loop(0, 16, zrow, 0)
        c0 = sid * nchunk // 16
        c1 = (sid + 1) * nchunk // 16
        for p in range(P):
            base_seg = pl.multiple_of((2 * p + core) * S, 8)
            row0 = pl.multiple_of(sid * rpt, 8)
            for (o, sz) in pieces:
                pltpu.sync_copy(z_v.at[pl.ds(0, sz)],
                                acc_sp.at[pl.ds(row0 + o, sz)])

            @pl.when(sid == 0)
            def _():
                pltpu.sync_copy(z_v.at[pl.ds(0, 16)], acc_sp.at[pl.ds(S, 16)])

            plsc.subcore_barrier()
            start(c0, 0)

            def chunk(ci, carry):
                par = lax.rem(ci - c0, 2)
                for b in (0, 1):
                    @pl.when(par == b)
                    def _():
                        @pl.when(ci > c0)
                        def _():
                            pltpu.make_async_copy(
                                co_v[1 - b], acc_sp.at[idxl_v[1 - b]],
                                asem[1 - b]).wait()

                        @pl.when(ci + 1 < c1)
                        def _():
                            start(ci + 1, 1 - b)
                        pltpu.make_async_copy(
                            dstidx.at[pl.ds(0, _CS)], idxd_v[b],
                            lsem[b]).wait()
                        pltpu.make_async_copy(
                            contrib.at[pl.ds(0, _CS)], co_v[b],
                            lsem[b]).wait()
                        for k8 in range(_CS // 16):
                            sl = pl.ds(k8 * 16, 16)
                            iv = idxd_v[b][sl] - base_seg
                            ok = (iv >= 0) & (iv < S)
                            idxl_v[b][sl] = jnp.where(ok, iv, S + sid)
                        pltpu.async_copy(co_v[b], acc_sp.at[idxl_v[b]],
                                        asem[b], add=True)
                return carry

            lax.fori_loop(c0, c1, chunk, 0)
            last_par = lax.rem(c1 - 1 - c0, 2)
            for b in (0, 1):
                @pl.when(last_par == b)
                def _():
                    pltpu.make_async_copy(co_v[b], acc_sp.at[idxl_v[b]],
                                          asem[b]).wait()
            plsc.subcore_barrier()
            for (o, sz) in pieces:
                pltpu.sync_copy(acc_sp.at[pl.ds(row0 + o, sz)],
                                w_v.at[pl.ds(0, sz)])
                pltpu.sync_copy(w_v.at[pl.ds(0, sz)],
                                out.at[pl.ds(pl.multiple_of(base_seg + row0 + o, 8), sz)])

    return k


_BS = 8192


def _take16(v, idx):
    dnums = lax.GatherDimensionNumbers(
        offset_dims=(), collapsed_slice_dims=(0,), start_index_map=(0,))
    return lax.gather(v, idx[:, None], dnums, slice_sizes=(1,),
                      mode=lax.GatherScatterMode.PROMISE_IN_BOUNDS)
_CAP = 1024


@functools.lru_cache(None)
def _partition_call(e, nbuk):
    nchunk = e // _C
    rows = _NW * nbuk * _CAP + 8
    trash = rows - 8
    big = 1 << 28
    mesh = plsc.VectorSubcoreMesh(core_axis_name="c", subcore_axis_name="s")
    out_type = [
        jax.ShapeDtypeStruct((rows, 128), jnp.float32),
        jax.ShapeDtypeStruct((_NW, nbuk * _CAP), jnp.int32),
        jax.ShapeDtypeStruct((_NW, 128), jnp.int32),
    ]
    scratch = [
        pltpu.VMEM((_C,), jnp.int32), pltpu.VMEM((_C,), jnp.int32),
        pltpu.VMEM((_C, 128), jnp.float32), pltpu.VMEM((_C, 128), jnp.float32),
        pltpu.VMEM((_C,), jnp.int32),
        pltpu.VMEM((nbuk * _CAP,), jnp.int32),
        pltpu.VMEM((128,), jnp.int32),
        pltpu.VMEM((128,), jnp.int32),
        pltpu.SemaphoreType.DMA, pltpu.SemaphoreType.DMA,
    ]

    @functools.partial(pl.kernel, mesh=mesh, out_type=out_type,
                       scratch_types=scratch)
    def k(contrib, dstidx, rout, bout, cout, i0, i1, cv0, cv1, gidx_v,
          stage_v, cflush_v, cnt_v, sm0, sm1):
        idxd_v = [i0, i1]
        co_v = [cv0, cv1]
        lsem = [sm0, sm1]
        wid = lax.axis_index("s") * 2 + lax.axis_index("c")
        c0 = wid * nchunk // _NW
        c1 = (wid + 1) * nchunk // _NW

        ones16 = jnp.ones((16,), jnp.int32)
        zeros16 = jnp.zeros((16,), jnp.int32)
        lanes = lax.iota(jnp.int32, 16)
        for j in range(8):
            cnt_v[pl.ds(j * 16, 16)] = zeros16
        for j in range(8):
            cflush_v[pl.ds(j * 16, 16)] = zeros16

        def initstage(j, carry):
            stage_v[pl.ds(j * 16, 16)] = zeros16 + big
            return carry

        lax.fori_loop(0, nbuk * _CAP // 16, initstage, 0)

        def start(ci, b):
            base = ci * _C
            pltpu.async_copy(dstidx.at[pl.ds(base, _C)], idxd_v[b], lsem[b])
            pltpu.async_copy(contrib.at[pl.ds(base, _C)], co_v[b], lsem[b])

        start(c0, 0)

        def chunk(ci, carry):
            par = lax.rem(ci - c0, 2)
            for b in (0, 1):
                @pl.when(par == b)
                def _():
                    @pl.when(ci + 1 < c1)
                    def _():
                        start(ci + 1, 1 - b)
                    pltpu.make_async_copy(
                        dstidx.at[pl.ds(0, _C)], idxd_v[b], lsem[b]).wait()
                    pltpu.make_async_copy(
                        contrib.at[pl.ds(0, _C)], co_v[b], lsem[b]).wait()
                    for k8 in range(8):
                        sl = pl.ds(k8 * 16, 16)
                        iv = idxd_v[b][sl]
                        bk = lax.shift_right_logical(iv, 13)
                        rank = zeros16
                        after = zeros16
                        for d in range(1, 16):
                            dn = _take16(bk, jnp.maximum(lanes - d, 0))
                            up = _take16(bk, jnp.minimum(lanes + d, 15))
                            rank = rank + jnp.where(
                                (dn == bk) & (lanes >= d), ones16, zeros16)
                            after = after + jnp.where(
                                (up == bk) & (lanes < 16 - d), ones16,
                                zeros16)
                        base_c = plsc.load_gather(cnt_v, [bk])
                        slot = base_c + rank
                        okc = slot < _CAP
                        plsc.store_scatter(cnt_v, [bk], slot + 1,
                                           mask=(after == 0))
                        g = wid * nbuk * _CAP + bk * _CAP + slot
                        gidx_v[sl] = jnp.where(okc, g, zeros16 + trash)
                        plsc.store_scatter(stage_v, [bk * _CAP + slot], iv,
                                           mask=okc)
                    pltpu.sync_copy(co_v[b], rout.at[gidx_v])
                return carry

        lax.fori_loop(c0, c1, chunk, 0)
        cflush_v[pl.ds(0, 16)] = cnt_v[pl.ds(0, 16)]
        cflush_v[pl.ds(16, 16)] = cnt_v[pl.ds(16, 16)]
        pltpu.sync_copy(stage_v, bout.at[wid])
        pltpu.sync_copy(cflush_v, cout.at[wid])

    return k


@functools.lru_cache(None)
def _bucket_acc_call(nbuk):
    sacc = _BS + 128
    rpt_z = sacc // 16
    rpt_w = _BS // 16
    mesh = plsc.VectorSubcoreMesh(core_axis_name="c", subcore_axis_name="s")
    scratch = [
        pltpu.VMEM((_C,), jnp.int32), pltpu.VMEM((_C,), jnp.int32),
        pltpu.VMEM((_C,), jnp.int32), pltpu.VMEM((_C,), jnp.int32),
        pltpu.VMEM((_C, 128), jnp.float32), pltpu.VMEM((_C, 128), jnp.float32),
        pltpu.VMEM((64, 128), jnp.float32),
        pltpu.VMEM((64, 128), jnp.float32),
        pltpu.VMEM((_NW, 128), jnp.int32),
        pltpu.VMEM_SHARED((sacc, 128), jnp.float32),
        pltpu.SemaphoreType.DMA, pltpu.SemaphoreType.DMA,
    ]

    @functools.partial(
        pl.kernel, mesh=mesh,
        out_type=jax.ShapeDtypeStruct((nbuk * _BS, 128), jnp.float32),
        scratch_types=scratch)
    def k(rowsbuf, bidx, counts, out, i0, i1, l0, l1, cv0, cv1, z_v, w_v,
          cnt_v, acc_sp, sm0, sm1):
        idxd_v = [i0, i1]
        idxl_v = [l0, l1]
        co_v = [cv0, cv1]
        lsem = [sm0, sm1]
        core = lax.axis_index("c")
        sid = lax.axis_index("s")
        pltpu.sync_copy(counts, cnt_v)

        def zrow(r, c2):
            for k4 in range(8):
                z_v[r, pl.ds(k4 * 16, 16)] = jnp.zeros((16,), jnp.float32)
            return c2

        lax.fori_loop(0, 64, zrow, 0)

        def start(regbase, j, b):
            base = regbase + j * _C
            pltpu.async_copy(rowsbuf.at[pl.ds(base, _C)], co_v[b], lsem[b])

        def startidx(pt, bu, j, b):
            pltpu.async_copy(
                bidx.at[pt, pl.ds(bu * _CAP + j * _C, _C)], idxd_v[b],
                lsem[b])

        for bu in range(nbuk):
            @pl.when(core == (bu % 2))
            def _():
                row0 = pl.multiple_of(sid * rpt_z, 8)
                for o in range(0, rpt_z, 64):
                    sz = min(64, rpt_z - o)
                    pltpu.sync_copy(z_v.at[pl.ds(0, sz)],
                                    acc_sp.at[pl.ds(row0 + o, sz)])
                plsc.subcore_barrier()
                for pt_off in (0, 16):
                    pt = sid + pt_off
                    cvec = cnt_v[pt, pl.ds((bu // 16) * 16, 16)]
                    cnt = cvec[bu % 16]
                    nch = jnp.minimum(
                        lax.shift_right_logical(cnt + (_C - 1), 7),
                        _CAP // _C)
                    regbase = (pt * nbuk + bu) * _CAP

                    @pl.when(nch > 0)
                    def _():
                        startidx(pt, bu, 0, 0)
                        start(regbase, 0, 0)

                        def chunk(j, carry):
                            par = lax.rem(j, 2)
                            for b in (0, 1):
                                @pl.when(par == b)
                                def _():
                                    @pl.when(j + 1 < nch)
                                    def _():
                                        startidx(pt, bu, j + 1, 1 - b)
                                        start(regbase, j + 1, 1 - b)
                                    pltpu.make_async_copy(
                                        bidx.at[0, pl.ds(0, _C)],
                                        idxd_v[b], lsem[b]).wait()
                                    pltpu.make_async_copy(
                                        rowsbuf.at[pl.ds(0, _C)],
                                        co_v[b], lsem[b]).wait()
                                    for k8 in range(8):
                                        sl = pl.ds(k8 * 16, 16)
                                        iv = idxd_v[b][sl] - bu * _BS
                                        ok = (iv >= 0) & (iv < _BS)
                                        idxl_v[b][sl] = jnp.where(
                                            ok, iv, _BS)
                                    pltpu.sync_copy(
                                        co_v[b], acc_sp.at[idxl_v[b]],
                                        add=True)
                            return carry

                        lax.fori_loop(0, nch, chunk, 0)
                plsc.subcore_barrier()
                wrow = pl.multiple_of(sid * rpt_w, 8)
                for o in range(0, rpt_w, 64):
                    sz = min(64, rpt_w - o)
                    pltpu.sync_copy(acc_sp.at[pl.ds(wrow + o, sz)],
                                    w_v.at[pl.ds(0, sz)])
                    pltpu.sync_copy(
                        w_v.at[pl.ds(0, sz)],
                        out.at[pl.ds(
                            pl.multiple_of(bu * _BS + wrow + o, 8), sz)])

    return k


def _bucket_scatter(contrib, dsti, n):
    e = contrib.shape[0]
    nbuk = (n + _BS - 1) // _BS
    rout, bout, cout = _partition_call(e, nbuk)(contrib, dsti)
    return _bucket_acc_call(nbuk)(rout, bout, cout)


def _egc_fused(p, srci, dsti, pieces_n, pieces_e, stats_of, n_nodes, e_edges,
               bucketed, write_m):
    stn = [stats_of(a) for a in pieces_n]
    W1 = jnp.concatenate([p['W_src_gate'], p['W_dst_update']], axis=1)
    W2 = jnp.concatenate([p['W_dst_gate'],
                          jnp.zeros_like(p['W_dst_gate'])], axis=1)
    tbl_src, tbl_dst, xwsu = _mm(pieces_n, stn, p['bn_n_g'], p['bn_n_b'],
                                 [W1, W2, p['W_src_update']])
    ste = [stats_of(a) for a in pieces_e]
    (ye,) = _mm(pieces_e, ste, p['bn_e_g'], p['bn_e_b'], [p['W_edge_gate']])
    ec = _edge_compute_call(e_edges, write_m)
    if write_m:
        m, contrib = ec(tbl_src, tbl_dst, ye, srci, dsti)
    else:
        out = ec(tbl_src, tbl_dst, ye, srci, dsti)
        contrib = out[0] if isinstance(out, (list, tuple)) else out
        m = None
    if bucketed:
        acc = _edge_scatter_call(e_edges, 13440, 6)(contrib, dsti)
    else:
        acc = _edge_scatter_call(e_edges, 5120, 1)(contrib, dsti)
    BR = 2000 if n_nodes % 2000 == 0 else 1000
    x_out, x_st = _combine_call(n_nodes, BR, acc.shape[0])(xwsu, acc)
    return x_out, x_st, m


def kernel(x, y, z, edge_index, lg_edge_index, params):
    src = edge_index[0].astype(jnp.int32)
    dst = edge_index[1].astype(jnp.int32)
    lsrc = lg_edge_index[0].astype(jnp.int32)
    ldst = lg_edge_index[1].astype(jnp.int32)
    N = x.shape[0]
    E = y.shape[0]
    T = z.shape[0]
    xs, ys, zs = [x], [y], [z]
    stats_cache = {}

    def stats_of(a):
        key = id(a)
        if key not in stats_cache:
            stats_cache[key] = _stats(a)
        return stats_cache[key]

    for i in range(_L):
        lp = params['layers'][i]
        nx, nx_st, ny = _egc_fused(lp['node_update'], src, dst, xs, ys,
                                   stats_of, N, E, bucketed=False,
                                   write_m=True)
        stats_cache[id(nx)] = nx_st
        ny2, ny2_st, nz = _egc_fused(lp['edge_update'], lsrc, ldst, [ny], zs,
                                     stats_of, E, T, bucketed=True,
                                     write_m=(i < _L - 1))
        stats_cache[id(ny2)] = ny2_st
        xs.append(nx)
        ys.append(ny2)
        if nz is not None:
            zs.append(nz)

    bx = params['bottleneck_x']
    by = params['bottleneck_y']
    (x_out,) = _mm(xs, [stats_of(a) for a in xs], bx['g'], bx['b'],
                   [bx['W']], residual=x)
    (y_out,) = _mm(ys, [stats_of(a) for a in ys], by['g'], by['b'],
                   [by['W']], residual=y)
    return (x_out, y_out)


# speedup vs baseline: 2.7937x; 1.2263x over previous
import functools

import jax
import jax.numpy as jnp
from jax import lax
from jax.experimental import pallas as pl
from jax.experimental.pallas import tpu as pltpu
from jax.experimental.pallas import tpu_sc as plsc

_EPS_BN = 1e-5
_EPS_SEG = 1e-6
_C = 128
_NW = 32
_L = 3


@functools.lru_cache(None)
def _stats_call(R, F, BR):
    def body(a_ref, o_ref):
        a = a_ref[...]

        @pl.when(pl.program_id(0) == 0)
        def _():
            o_ref[...] = jnp.zeros_like(o_ref)

        o_ref[0:1, :] += jnp.sum(a, axis=0, keepdims=True)
        o_ref[1:2, :] += jnp.sum(a * a, axis=0, keepdims=True)

    return pl.pallas_call(
        body,
        grid=(R // BR,),
        in_specs=[pl.BlockSpec((BR, F), lambda i: (i, 0))],
        out_specs=pl.BlockSpec((8, F), lambda i: (0, 0)),
        out_shape=jax.ShapeDtypeStruct((8, F), jnp.float32),
    )


def _stats(a):
    R, F = a.shape
    BR = 2000 if R % 2000 == 0 else 1000
    return _stats_call(R, F, BR)(a)


@functools.lru_cache(None)
def _mm_call(R, BR, npieces, Ks, residual):
    nouts = len(Ks)

    def body(*refs):
        it = iter(refs)
        a = [next(it) for _ in range(npieces)]
        st = [next(it) for _ in range(npieces)]
        g = [next(it) for _ in range(npieces)]
        b = [next(it) for _ in range(npieces)]
        W = [[next(it) for _ in range(npieces)] for _ in range(nouts)]
        res = next(it) if residual else None
        outs = [next(it) for _ in range(nouts)]
        acc = [None] * nouts
        inv_r = 1.0 / R
        for j in range(npieces):
            aj = a[j][...]
            mean = st[j][0:1, :] * inv_r
            var = st[j][1:2, :] * inv_r - mean * mean
            xn = (aj - mean) * lax.rsqrt(var + _EPS_BN) * g[j][...] + b[j][...]
            xn = xn * jax.nn.sigmoid(xn)
            for o in range(nouts):
                d = jnp.dot(xn, W[o][j][...], preferred_element_type=jnp.float32)
                acc[o] = d if acc[o] is None else acc[o] + d
        for o in range(nouts):
            val = acc[o]
            if residual and o == 0:
                val = val + res[...]
            outs[o][...] = val

    in_specs = (
        [pl.BlockSpec((BR, 64), lambda i: (i, 0)) for _ in range(npieces)]
        + [pl.BlockSpec((8, 64), lambda i: (0, 0)) for _ in range(npieces)]
        + [pl.BlockSpec((1, 64), lambda i: (0, 0)) for _ in range(2 * npieces)]
        + [pl.BlockSpec((64, K), lambda i: (0, 0))
           for K in Ks for _ in range(npieces)]
        + ([pl.BlockSpec((BR, Ks[0]), lambda i: (i, 0))] if residual else [])
    )
    return pl.pallas_call(
        body,
        grid=(R // BR,),
        in_specs=in_specs,
        out_specs=[pl.BlockSpec((BR, K), lambda i: (i, 0)) for K in Ks],
        out_shape=[jax.ShapeDtypeStruct((R, K), jnp.float32) for K in Ks],
    )


def _mm(pieces, stats, g, b, Ws, residual=None):
    R = pieces[0].shape[0]
    npieces = len(pieces)
    Ks = tuple(int(W.shape[1]) for W in Ws)
    BR = 2000 if R % 2000 == 0 else 1000
    args = list(pieces) + list(stats)
    for j in range(npieces):
        args.append(g[64 * j:64 * (j + 1)].reshape(1, 64))
    for j in range(npieces):
        args.append(b[64 * j:64 * (j + 1)].reshape(1, 64))
    for W in Ws:
        for j in range(npieces):
            args.append(W[64 * j:64 * (j + 1), :])
    if residual is not None:
        args.append(residual)
    out = _mm_call(R, BR, npieces, Ks, residual is not None)(*args)
    return out


@functools.lru_cache(None)
def _combine_call(R, BR, Racc):
    def body(su_ref, acc_ref, o_ref, st_ref):
        accv = acc_ref[...]
        nx = su_ref[...] + accv[:, 0:64] / (accv[:, 64:128] + _EPS_SEG)
        o_ref[...] = nx

        @pl.when(pl.program_id(0) == 0)
        def _():
            st_ref[...] = jnp.zeros_like(st_ref)

        st_ref[0:1, :] += jnp.sum(nx, axis=0, keepdims=True)
        st_ref[1:2, :] += jnp.sum(nx * nx, axis=0, keepdims=True)

    return pl.pallas_call(
        body,
        grid=(R // BR,),
        in_specs=[pl.BlockSpec((BR, 64), lambda i: (i, 0)),
                  pl.BlockSpec((BR, 128), lambda i: (i, 0))],
        out_specs=[pl.BlockSpec((BR, 64), lambda i: (i, 0)),
                   pl.BlockSpec((8, 64), lambda i: (0, 0))],
        out_shape=[jax.ShapeDtypeStruct((R, 64), jnp.float32),
                   jax.ShapeDtypeStruct((8, 64), jnp.float32)],
    )


@functools.lru_cache(None)
def _edge_compute_call(e, write_m):
    CE = 64
    nchunk = e // CE
    mesh = plsc.VectorSubcoreMesh(core_axis_name="c", subcore_axis_name="s")
    out_type = [jax.ShapeDtypeStruct((e, 128), jnp.float32)]
    if write_m:
        out_type = [jax.ShapeDtypeStruct((e, 64), jnp.float32)] + out_type
    scratch = [
        pltpu.VMEM((CE,), jnp.int32), pltpu.VMEM((CE,), jnp.int32),
        pltpu.VMEM((CE,), jnp.int32), pltpu.VMEM((CE,), jnp.int32),
        pltpu.VMEM((CE, 128), jnp.float32), pltpu.VMEM((CE, 128), jnp.float32),
        pltpu.VMEM((CE, 128), jnp.float32), pltpu.VMEM((CE, 128), jnp.float32),
        pltpu.VMEM((CE, 64), jnp.float32), pltpu.VMEM((CE, 64), jnp.float32),
        pltpu.VMEM((CE, 64), jnp.float32), pltpu.VMEM((CE, 64), jnp.float32),
        pltpu.VMEM((CE, 128), jnp.float32), pltpu.VMEM((CE, 128), jnp.float32),
        pltpu.SemaphoreType.DMA, pltpu.SemaphoreType.DMA,
        pltpu.SemaphoreType.DMA, pltpu.SemaphoreType.DMA,
        pltpu.SemaphoreType.DMA, pltpu.SemaphoreType.DMA,
    ]

    @functools.partial(pl.kernel, mesh=mesh, out_type=out_type,
                       scratch_types=scratch)
    def k(tbl_src, tbl_dst, ef, src, dst, *rest):
        ri = iter(rest)
        m_out = next(ri) if write_m else None
        c_out = next(ri)
        idxs_v = [next(ri), next(ri)]
        idxd_v = [next(ri), next(ri)]
        gsrc_v = [next(ri), next(ri)]
        gdst_v = [next(ri), next(ri)]
        ef_v = [next(ri), next(ri)]
        m_v = [next(ri), next(ri)]
        co_v = [next(ri), next(ri)]
        gsem = [next(ri), next(ri)]
        isem = [next(ri), next(ri)]
        wsem = [next(ri), next(ri)]
        wid = lax.axis_index("s") * 2 + lax.axis_index("c")
        c0 = wid * nchunk // _NW
        c1 = (wid + 1) * nchunk // _NW

        def start_idx(ci, b):
            base = ci * CE
            pltpu.async_copy(src.at[pl.ds(base, CE)], idxs_v[b], isem[b])
            pltpu.async_copy(dst.at[pl.ds(base, CE)], idxd_v[b], isem[b])

        def wait_idx(b):
            pltpu.make_async_copy(src.at[pl.ds(0, CE)], idxs_v[b],
                                  isem[b]).wait()
            pltpu.make_async_copy(dst.at[pl.ds(0, CE)], idxd_v[b],
                                  isem[b]).wait()

        def start_gather(ci, b):
            base = ci * CE
            pltpu.async_copy(tbl_src.at[idxs_v[b]], gsrc_v[b], gsem[b])
            pltpu.async_copy(tbl_dst.at[idxd_v[b]], gdst_v[b], gsem[b])
            pltpu.async_copy(ef.at[pl.ds(base, CE)], ef_v[b], gsem[b])

        def wait_gather(b):
            pltpu.make_async_copy(tbl_src.at[idxs_v[b]], gsrc_v[b],
                                  gsem[b]).wait()
            pltpu.make_async_copy(tbl_dst.at[idxd_v[b]], gdst_v[b],
                                  gsem[b]).wait()
            pltpu.make_async_copy(ef.at[pl.ds(0, CE)], ef_v[b],
                                  gsem[b]).wait()

        def wait_write(b):
            if write_m:
                pltpu.make_async_copy(m_v[b], m_out.at[pl.ds(0, CE)],
                                      wsem[b]).wait()
            pltpu.make_async_copy(co_v[b], c_out.at[pl.ds(0, CE)],
                                  wsem[b]).wait()

        start_idx(c0, 0)
        start_idx(c0 + 1, 1)
        wait_idx(0)
        start_gather(c0, 0)

        def chunk(ci, carry):
            par = lax.rem(ci - c0, 2)
            for b in (0, 1):
                @pl.when(par == b)
                def _():
                    @pl.when(ci + 1 < c1)
                    def _():
                        wait_idx(1 - b)
                        start_gather(ci + 1, 1 - b)
                    wait_gather(b)

                    @pl.when(ci + 2 < c1)
                    def _():
                        start_idx(ci + 2, b)

                    @pl.when(ci - 2 >= c0)
                    def _():
                        wait_write(b)

                    def row(r, c2):
                        for k4 in range(4):
                            sl = pl.ds(k4 * 16, 16)
                            sh = pl.ds(64 + k4 * 16, 16)
                            mv = (gsrc_v[b][r, sl] + gdst_v[b][r, sl]
                                  + ef_v[b][r, sl])
                            sg = 1.0 / (1.0 + jnp.exp(-mv))
                            m_v[b][r, sl] = mv
                            co_v[b][r, sl] = gsrc_v[b][r, sh] * sg
                            co_v[b][r, sh] = sg
                        return c2

                    lax.fori_loop(0, CE, row, 0)
                    base = ci * CE
                    if write_m:
                        pltpu.async_copy(m_v[b], m_out.at[pl.ds(base, CE)],
                                         wsem[b])
                    pltpu.async_copy(co_v[b], c_out.at[pl.ds(base, CE)],
                                     wsem[b])
            return carry

        lax.fori_loop(c0, c1, chunk, 0)
        last_par = lax.rem(c1 - 1 - c0, 2)
        for b in (0, 1):
            @pl.when(last_par == b)
            def _():
                wait_write(b)

            @pl.when((c1 - c0 >= 2) & (last_par == 1 - b))
            def _():
                wait_write(b)


    return k


_CS = 64


@functools.lru_cache(None)
def _edge_scatter_call(e, S, P):
    nchunk = e // _CS
    npad = 2 * P * S
    rpt = S // 16
    pieces = []
    off = 0
    while off < rpt:
        sz = min(16, rpt - off)
        pieces.append((off, sz))
        off += sz
    mesh = plsc.VectorSubcoreMesh(core_axis_name="c", subcore_axis_name="s")
    scratch = [
        pltpu.VMEM((_CS,), jnp.int32), pltpu.VMEM((_CS,), jnp.int32),
        pltpu.VMEM((_CS,), jnp.int32), pltpu.VMEM((_CS,), jnp.int32),
        pltpu.VMEM((_CS, 128), jnp.float32),
        pltpu.VMEM((_CS, 128), jnp.float32),
        pltpu.VMEM((16, 128), jnp.float32),
        pltpu.VMEM((16, 128), jnp.float32),
        pltpu.VMEM_SHARED((S + 16, 128), jnp.float32),
        pltpu.SemaphoreType.DMA, pltpu.SemaphoreType.DMA,
        pltpu.SemaphoreType.DMA, pltpu.SemaphoreType.DMA,
    ]

    @functools.partial(pl.kernel, mesh=mesh,
                       out_type=jax.ShapeDtypeStruct((npad, 128), jnp.float32),
                       scratch_types=scratch)
    def k(contrib, dstidx, out, i0, i1, l0, l1, cv0, cv1, z_v, w_v, acc_sp,
          sm0, sm1, am0, am1):
        idxd_v = [i0, i1]
        idxl_v = [l0, l1]
        co_v = [cv0, cv1]
        lsem = [sm0, sm1]
        asem = [am0, am1]
        core = lax.axis_index("c")
        sid = lax.axis_index("s")

        def start(ci, b):
            base = ci * _CS
            pltpu.async_copy(dstidx.at[pl.ds(base, _CS)], idxd_v[b], lsem[b])
            pltpu.async_copy(contrib.at[pl.ds(base, _CS)], co_v[b], lsem[b])

        def zrow(r, c2):
            for k4 in range(8):
                z_v[r, pl.ds(k4 * 16, 16)] = jnp.zeros((16,), jnp.float32)
            return c2

        lax.fori_loop(0, 16, zrow, 0)
        c0 = sid * nchunk // 16
        c1 = (sid + 1) * nchunk // 16
        for p in range(P):
            base_seg = pl.multiple_of((2 * p + core) * S, 8)
            row0 = pl.multiple_of(sid * rpt, 8)
            for (o, sz) in pieces:
                pltpu.sync_copy(z_v.at[pl.ds(0, sz)],
                                acc_sp.at[pl.ds(row0 + o, sz)])

            @pl.when(sid == 0)
            def _():
                pltpu.sync_copy(z_v.at[pl.ds(0, 16)], acc_sp.at[pl.ds(S, 16)])

            plsc.subcore_barrier()
            start(c0, 0)

            def chunk(ci, carry):
                par = lax.rem(ci - c0, 2)
                for b in (0, 1):
                    @pl.when(par == b)
                    def _():
                        @pl.when(ci > c0)
                        def _():
                            pltpu.make_async_copy(
                                co_v[1 - b], acc_sp.at[idxl_v[1 - b]],
                                asem[1 - b]).wait()

                        @pl.when(ci + 1 < c1)
                        def _():
                            start(ci + 1, 1 - b)
                        pltpu.make_async_copy(
                            dstidx.at[pl.ds(0, _CS)], idxd_v[b],
                            lsem[b]).wait()
                        pltpu.make_async_copy(
                            contrib.at[pl.ds(0, _CS)], co_v[b],
                            lsem[b]).wait()
                        for k8 in range(_CS // 16):
                            sl = pl.ds(k8 * 16, 16)
                            iv = idxd_v[b][sl] - base_seg
                            ok = (iv >= 0) & (iv < S)
                            idxl_v[b][sl] = jnp.where(ok, iv, S + sid)
                        pltpu.async_copy(co_v[b], acc_sp.at[idxl_v[b]],
                                        asem[b], add=True)
                return carry

            lax.fori_loop(c0, c1, chunk, 0)
            last_par = lax.rem(c1 - 1 - c0, 2)
            for b in (0, 1):
                @pl.when(last_par == b)
                def _():
                    pltpu.make_async_copy(co_v[b], acc_sp.at[idxl_v[b]],
                                          asem[b]).wait()
            plsc.subcore_barrier()
            for (o, sz) in pieces:
                pltpu.sync_copy(acc_sp.at[pl.ds(row0 + o, sz)],
                                w_v.at[pl.ds(0, sz)])
                pltpu.sync_copy(w_v.at[pl.ds(0, sz)],
                                out.at[pl.ds(pl.multiple_of(base_seg + row0 + o, 8), sz)])

    return k


_BS = 8192


def _take16(v, idx):
    dnums = lax.GatherDimensionNumbers(
        offset_dims=(), collapsed_slice_dims=(0,), start_index_map=(0,))
    return lax.gather(v, idx[:, None], dnums, slice_sizes=(1,),
                      mode=lax.GatherScatterMode.PROMISE_IN_BOUNDS)
_CAP = 1024


@functools.lru_cache(None)
def _partition_call(e, nbuk):
    nchunk = e // _C
    rows = _NW * nbuk * _CAP + 8
    trash = rows - 8
    big = 1 << 28
    mesh = plsc.VectorSubcoreMesh(core_axis_name="c", subcore_axis_name="s")
    out_type = [
        jax.ShapeDtypeStruct((rows, 128), jnp.float32),
        jax.ShapeDtypeStruct((_NW, nbuk * _CAP), jnp.int32),
        jax.ShapeDtypeStruct((_NW, 128), jnp.int32),
    ]
    scratch = [
        pltpu.VMEM((_C,), jnp.int32), pltpu.VMEM((_C,), jnp.int32),
        pltpu.VMEM((_C, 128), jnp.float32), pltpu.VMEM((_C, 128), jnp.float32),
        pltpu.VMEM((_C,), jnp.int32),
        pltpu.VMEM((nbuk * _CAP,), jnp.int32),
        pltpu.VMEM((128,), jnp.int32),
        pltpu.VMEM((128,), jnp.int32),
        pltpu.SemaphoreType.DMA, pltpu.SemaphoreType.DMA,
    ]

    @functools.partial(pl.kernel, mesh=mesh, out_type=out_type,
                       scratch_types=scratch)
    def k(contrib, dstidx, rout, bout, cout, i0, i1, cv0, cv1, gidx_v,
          stage_v, cflush_v, cnt_v, sm0, sm1):
        idxd_v = [i0, i1]
        co_v = [cv0, cv1]
        lsem = [sm0, sm1]
        wid = lax.axis_index("s") * 2 + lax.axis_index("c")
        c0 = wid * nchunk // _NW
        c1 = (wid + 1) * nchunk // _NW

        ones16 = jnp.ones((16,), jnp.int32)
        zeros16 = jnp.zeros((16,), jnp.int32)
        lanes = lax.iota(jnp.int32, 16)
        for j in range(8):
            cnt_v[pl.ds(j * 16, 16)] = zeros16
        for j in range(8):
            cflush_v[pl.ds(j * 16, 16)] = zeros16

        def initstage(j, carry):
            stage_v[pl.ds(j * 16, 16)] = zeros16 + big
            return carry

        lax.fori_loop(0, nbuk * _CAP // 16, initstage, 0)

        def start(ci, b):
            base = ci * _C
            pltpu.async_copy(dstidx.at[pl.ds(base, _C)], idxd_v[b], lsem[b])
            pltpu.async_copy(contrib.at[pl.ds(base, _C)], co_v[b], lsem[b])

        start(c0, 0)

        def chunk(ci, carry):
            par = lax.rem(ci - c0, 2)
            for b in (0, 1):
                @pl.when(par == b)
                def _():
                    @pl.when(ci + 1 < c1)
                    def _():
                        start(ci + 1, 1 - b)
                    pltpu.make_async_copy(
                        dstidx.at[pl.ds(0, _C)], idxd_v[b], lsem[b]).wait()
                    pltpu.make_async_copy(
                        contrib.at[pl.ds(0, _C)], co_v[b], lsem[b]).wait()
                    for k8 in range(8):
                        sl = pl.ds(k8 * 16, 16)
                        iv = idxd_v[b][sl]
                        bk = lax.shift_right_logical(iv, 13)
                        rank = zeros16
                        after = zeros16
                        for d in range(1, 16):
                            dn = _take16(bk, jnp.maximum(lanes - d, 0))
                            up = _take16(bk, jnp.minimum(lanes + d, 15))
                            rank = rank + jnp.where(
                                (dn == bk) & (lanes >= d), ones16, zeros16)
                            after = after + jnp.where(
                                (up == bk) & (lanes < 16 - d), ones16,
                                zeros16)
                        base_c = plsc.load_gather(cnt_v, [bk])
                        slot = base_c + rank
                        okc = slot < _CAP
                        plsc.store_scatter(cnt_v, [bk], slot + 1,
                                           mask=(after == 0))
                        g = wid * nbuk * _CAP + bk * _CAP + slot
                        gidx_v[sl] = jnp.where(okc, g, zeros16 + trash)
                        plsc.store_scatter(stage_v, [bk * _CAP + slot], iv,
                                           mask=okc)
                    pltpu.sync_copy(co_v[b], rout.at[gidx_v])
                return carry

        lax.fori_loop(c0, c1, chunk, 0)
        cflush_v[pl.ds(0, 16)] = cnt_v[pl.ds(0, 16)]
        cflush_v[pl.ds(16, 16)] = cnt_v[pl.ds(16, 16)]
        pltpu.sync_copy(stage_v, bout.at[wid])
        pltpu.sync_copy(cflush_v, cout.at[wid])

    return k


@functools.lru_cache(None)
def _bucket_acc_call(nbuk):
    sacc = _BS + 128
    rpt_z = sacc // 16
    rpt_w = _BS // 16
    mesh = plsc.VectorSubcoreMesh(core_axis_name="c", subcore_axis_name="s")
    scratch = [
        pltpu.VMEM((_C,), jnp.int32), pltpu.VMEM((_C,), jnp.int32),
        pltpu.VMEM((_C,), jnp.int32), pltpu.VMEM((_C,), jnp.int32),
        pltpu.VMEM((_C, 128), jnp.float32), pltpu.VMEM((_C, 128), jnp.float32),
        pltpu.VMEM((64, 128), jnp.float32),
        pltpu.VMEM((64, 128), jnp.float32),
        pltpu.VMEM((_NW, 128), jnp.int32),
        pltpu.VMEM_SHARED((sacc, 128), jnp.float32),
        pltpu.SemaphoreType.DMA, pltpu.SemaphoreType.DMA,
    ]

    @functools.partial(
        pl.kernel, mesh=mesh,
        out_type=jax.ShapeDtypeStruct((nbuk * _BS, 128), jnp.float32),
        scratch_types=scratch)
    def k(rowsbuf, bidx, counts, out, i0, i1, l0, l1, cv0, cv1, z_v, w_v,
          cnt_v, acc_sp, sm0, sm1):
        idxd_v = [i0, i1]
        idxl_v = [l0, l1]
        co_v = [cv0, cv1]
        lsem = [sm0, sm1]
        core = lax.axis_index("c")
        sid = lax.axis_index("s")
        pltpu.sync_copy(counts, cnt_v)

        def zrow(r, c2):
            for k4 in range(8):
                z_v[r, pl.ds(k4 * 16, 16)] = jnp.zeros((16,), jnp.float32)
            return c2

        lax.fori_loop(0, 64, zrow, 0)

        def start(regbase, j, b):
            base = regbase + j * _C
            pltpu.async_copy(rowsbuf.at[pl.ds(base, _C)], co_v[b], lsem[b])

        def startidx(pt, bu, j, b):
            pltpu.async_copy(
                bidx.at[pt, pl.ds(bu * _CAP + j * _C, _C)], idxd_v[b],
                lsem[b])

        for bu in range(nbuk):
            @pl.when(core == (bu % 2))
            def _():
                row0 = pl.multiple_of(sid * rpt_z, 8)
                for o in range(0, rpt_z, 64):
                    sz = min(64, rpt_z - o)
                    pltpu.sync_copy(z_v.at[pl.ds(0, sz)],
                                    acc_sp.at[pl.ds(row0 + o, sz)])
                plsc.subcore_barrier()
                for pt_off in (0, 16):
                    pt = sid + pt_off
                    cvec = cnt_v[pt, pl.ds((bu // 16) * 16, 16)]
                    cnt = cvec[bu % 16]
                    nch = jnp.minimum(
                        lax.shift_right_logical(cnt + (_C - 1), 7),
                        _CAP // _C)
                    regbase = (pt * nbuk + bu) * _CAP

                    @pl.when(nch > 0)
                    def _():
                        startidx(pt, bu, 0, 0)
                        start(regbase, 0, 0)

                        def chunk(j, carry):
                            par = lax.rem(j, 2)
                            for b in (0, 1):
                                @pl.when(par == b)
                                def _():
                                    @pl.when(j + 1 < nch)
                                    def _():
                                        startidx(pt, bu, j + 1, 1 - b)
                                        start(regbase, j + 1, 1 - b)
                                    pltpu.make_async_copy(
                                        bidx.at[0, pl.ds(0, _C)],
                                        idxd_v[b], lsem[b]).wait()
                                    pltpu.make_async_copy(
                                        rowsbuf.at[pl.ds(0, _C)],
                                        co_v[b], lsem[b]).wait()
                                    for k8 in range(8):
                                        sl = pl.ds(k8 * 16, 16)
                                        iv = idxd_v[b][sl] - bu * _BS
                                        ok = (iv >= 0) & (iv < _BS)
                                        idxl_v[b][sl] = jnp.where(
                                            ok, iv, _BS)
                                    pltpu.sync_copy(
                                        co_v[b], acc_sp.at[idxl_v[b]],
                                        add=True)
                            return carry

                        lax.fori_loop(0, nch, chunk, 0)
                plsc.subcore_barrier()
                wrow = pl.multiple_of(sid * rpt_w, 8)
                for o in range(0, rpt_w, 64):
                    sz = min(64, rpt_w - o)
                    pltpu.sync_copy(acc_sp.at[pl.ds(wrow + o, sz)],
                                    w_v.at[pl.ds(0, sz)])
                    pltpu.sync_copy(
                        w_v.at[pl.ds(0, sz)],
                        out.at[pl.ds(
                            pl.multiple_of(bu * _BS + wrow + o, 8), sz)])

    return k


def _bucket_scatter(contrib, dsti, n):
    e = contrib.shape[0]
    nbuk = (n + _BS - 1) // _BS
    rout, bout, cout = _partition_call(e, nbuk)(contrib, dsti)
    return _bucket_acc_call(nbuk)(rout, bout, cout)


def _egc_fused(p, srci, dsti, pieces_n, pieces_e, stats_of, n_nodes, e_edges,
               bucketed, write_m):
    stn = [stats_of(a) for a in pieces_n]
    W1 = jnp.concatenate([p['W_src_gate'], p['W_dst_update']], axis=1)
    W2 = jnp.concatenate([p['W_dst_gate'],
                          jnp.zeros_like(p['W_dst_gate'])], axis=1)
    tbl_src, tbl_dst, xwsu = _mm(pieces_n, stn, p['bn_n_g'], p['bn_n_b'],
                                 [W1, W2, p['W_src_update']])
    ste = [stats_of(a) for a in pieces_e]
    (ye,) = _mm(pieces_e, ste, p['bn_e_g'], p['bn_e_b'], [p['W_edge_gate']])
    ec = _edge_compute_call(e_edges, write_m)
    if write_m:
        m, contrib = ec(tbl_src, tbl_dst, ye, srci, dsti)
    else:
        out = ec(tbl_src, tbl_dst, ye, srci, dsti)
        contrib = out[0] if isinstance(out, (list, tuple)) else out
        m = None
    if bucketed:
        acc = _edge_scatter_call(e_edges, 13440, 6)(contrib, dsti)
    else:
        acc = _edge_scatter_call(e_edges, 5120, 1)(contrib, dsti)
    BR = 2000 if n_nodes % 2000 == 0 else 1000
    x_out, x_st = _combine_call(n_nodes, BR, acc.shape[0])(xwsu, acc)
    return x_out, x_st, m


def kernel(x, y, z, edge_index, lg_edge_index, params):
    src = edge_index[0].astype(jnp.int32)
    dst = edge_index[1].astype(jnp.int32)
    lsrc = lg_edge_index[0].astype(jnp.int32)
    ldst = lg_edge_index[1].astype(jnp.int32)
    N = x.shape[0]
    E = y.shape[0]
    T = z.shape[0]
    xs, ys, zs = [x], [y], [z]
    stats_cache = {}

    def stats_of(a):
        key = id(a)
        if key not in stats_cache:
            stats_cache[key] = _stats(a)
        return stats_cache[key]

    for i in range(_L):
        lp = params['layers'][i]
        nx, nx_st, ny = _egc_fused(lp['node_update'], src, dst, xs, ys,
                                   stats_of, N, E, bucketed=False,
                                   write_m=True)
        stats_cache[id(nx)] = nx_st
        ny2, ny2_st, nz = _egc_fused(lp['edge_update'], lsrc, ldst, [ny], zs,
                                     stats_of, E, T, bucketed=True,
                                     write_m=(i < _L - 1))
        stats_cache[id(ny2)] = ny2_st
        xs.append(nx)
        ys.append(ny2)
        if nz is not None:
            zs.append(nz)

    bx = params['bottleneck_x']
    by = params['bottleneck_y']
    (x_out,) = _mm(xs, [stats_of(a) for a in xs], bx['g'], bx['b'],
                   [bx['W']], residual=x)
    (y_out,) = _mm(ys, [stats_of(a) for a in ys], by['g'], by['b'],
                   [by['W']], residual=y)
    return (x_out, y_out)
